# Initial kernel scaffold; baseline (speedup 1.0000x reference)
import jax
import jax.numpy as jnp
from jax.experimental import pallas as pl


def kernel(pred_logits, pred_boxes, tgt_boxes, tgt_labels, tgt_size):
    raise NotImplementedError("write your pallas kernel here")


# speedup vs baseline: 47.3568x; 47.3568x over previous
import functools

import jax
import jax.numpy as jnp
from jax import lax
from jax.experimental import pallas as pl
from jax.experimental.pallas import tpu as pltpu
from jax.experimental.pallas import tpu_sc as plsc

NCLS = 15
CLS_W = 2.0
BBOX_W = 5.0
ANG_W = 2.0
NOOBJ_W = 0.1
B, Q, M = 4, 1000, 200
QP, MP = 1024, 256
LANES = 16
BIGI = 2 ** 30

_COS_C = [
    1.0, -0.5, 1.0 / 24, -1.0 / 720, 1.0 / 40320, -1.0 / 3628800,
    1.0 / 479001600, -1.0 / 87178291200,
]


def _cos_scalar(x):
    t = x * x
    r = jnp.float32(_COS_C[7])
    for k in range(6, -1, -1):
        r = r * t + jnp.float32(_COS_C[k])
    return r


def _tc_body(size_ref, lt_ref, pbt_ref, tb_ref, lab_ref,
             cost_ref, colmin_ref, colargq_ref, logp_ref, base_ref):
    lt = lt_ref[0]
    mx = jnp.max(lt, axis=0, keepdims=True)
    ex = jnp.exp(lt - mx)
    s = jnp.sum(ex, axis=0, keepdims=True)
    logp = lt - mx - jnp.log(s)
    logp_ref[0] = logp
    prob = ex / s

    hh = size_ref[0, 0, 0].astype(jnp.float32)
    ww = size_ref[0, 0, 1].astype(jnp.float32)
    tb = tb_ref[0]
    lab0 = lab_ref[0] - 1

    cls = jnp.zeros((MP, QP), jnp.float32)
    for c in range(NCLS):
        cls = cls + jnp.where(lab0 == c, prob[c:c + 1, :], 0.0)
    cost = cls * (-CLS_W)

    scales = (ww, hh, ww, hh)
    for d in range(4):
        nd = tb[:, d:d + 1] / scales[d]
        cost = cost + BBOX_W * jnp.abs(pbt_ref[0, d:d + 1, :] - nd)
    cost = cost + ANG_W * (1.0 - jnp.cos(pbt_ref[0, 4:5, :] - tb[:, 4:5]))

    qi = lax.broadcasted_iota(jnp.int32, (MP, QP), 1)
    mi = lax.broadcasted_iota(jnp.int32, (MP, QP), 0)
    cost = jnp.where((qi >= Q) | (mi >= M), jnp.inf, cost)
    cost_ref[0] = cost

    cmin = jnp.min(cost, axis=1, keepdims=True)
    colmin_ref[0] = cmin
    ismin = cost == cmin
    colargq_ref[0] = jnp.min(jnp.where(ismin, qi, QP), axis=1, keepdims=True)

    row15 = logp[NCLS:NCLS + 1, :]
    qrow = lax.broadcasted_iota(jnp.int32, (1, QP), 1)
    base_ref[0, 0, 0] = NOOBJ_W * jnp.sum(jnp.where(qrow < Q, -row15, 0.0))


def _tc_stage(size, lt, pbt, tbp, labp):
    f32 = jnp.float32
    out_shapes = (
        jax.ShapeDtypeStruct((B, MP, QP), f32),
        jax.ShapeDtypeStruct((B, MP, 1), f32),
        jax.ShapeDtypeStruct((B, MP, 1), jnp.int32),
        jax.ShapeDtypeStruct((B, 16, QP), f32),
        jax.ShapeDtypeStruct((B, 1, 1), f32),
    )
    grid = (B,)
    return pl.pallas_call(
        _tc_body,
        grid=grid,
        in_specs=[
            pl.BlockSpec((1, 1, 2), lambda b: (b, 0, 0), memory_space=pltpu.SMEM),
            pl.BlockSpec((1, 16, QP), lambda b: (b, 0, 0)),
            pl.BlockSpec((1, 5, QP), lambda b: (b, 0, 0)),
            pl.BlockSpec((1, MP, 5), lambda b: (b, 0, 0)),
            pl.BlockSpec((1, MP, 1), lambda b: (b, 0, 0)),
        ],
        out_specs=[
            pl.BlockSpec((1, MP, QP), lambda b: (b, 0, 0)),
            pl.BlockSpec((1, MP, 1), lambda b: (b, 0, 0)),
            pl.BlockSpec((1, MP, 1), lambda b: (b, 0, 0)),
            pl.BlockSpec((1, 16, QP), lambda b: (b, 0, 0)),
            pl.BlockSpec((1, 1, 1), lambda b: (b, 0, 0), memory_space=pltpu.SMEM),
        ],
        out_shape=out_shapes,
    )(size, lt, pbt, tbp, labp)


def _sc_greedy(cost_hbm, colmin_hbm, colargq_hbm, logp_hbm, pbt_hbm,
               tb_hbm, lab_hbm, size_hbm, out_hbm,
               colmin_v, colargq_v, logp_v, pbt_v, tb_v, lab_v, size_v,
               qmask_v, rowbuf_v, outbuf_v):
    info = plsc.get_sparse_core_info()
    nc = info.num_cores
    wid = lax.axis_index("s") * nc + lax.axis_index("c")

    iota16 = lax.broadcasted_iota(jnp.int32, (LANES,), 0)
    lane0 = iota16 == 0

    def _gat(ref, *idx):
        idxs = [jnp.broadcast_to(i, (LANES,)).astype(jnp.int32) for i in idx]
        return plsc.load_gather(ref, idxs)[0]

    def _put(ref, i, val):
        ii = jnp.broadcast_to(i, (LANES,)).astype(jnp.int32)
        plsc.store_scatter(ref, [ii], jnp.broadcast_to(val, (LANES,)),
                           mask=lane0)

    @pl.when(wid < B)
    def _work():
        b = wid
        pltpu.sync_copy(colmin_hbm.at[b], colmin_v)
        pltpu.sync_copy(colargq_hbm.at[b], colargq_v)
        pltpu.sync_copy(logp_hbm.at[b], logp_v)
        pltpu.sync_copy(pbt_hbm.at[b], pbt_v)
        pltpu.sync_copy(tb_hbm.at[b], tb_v)
        pltpu.sync_copy(lab_hbm.at[b], lab_v)
        pltpu.sync_copy(size_hbm.at[b], size_v)

        zeros16 = jnp.zeros((LANES,), jnp.float32)
        for k in range(QP // LANES):
            qmask_v[pl.ds(k * LANES, LANES)] = zeros16

        sizes = size_v[pl.ds(0, LANES)]
        rcp = 1.0 / sizes.astype(jnp.float32)
        rw = rcp[1]
        rh = rcp[0]
        rs = (rw, rh, rw, rh)
        inf = jnp.float32(jnp.inf)

        def recompute_col(m2):
            pltpu.sync_copy(cost_hbm.at[b, m2], rowbuf_v)
            bv = rowbuf_v[pl.ds(0, LANES)] + qmask_v[pl.ds(0, LANES)]
            bq = iota16
            for k in range(1, QP // LANES):
                v = rowbuf_v[pl.ds(k * LANES, LANES)] + qmask_v[pl.ds(k * LANES, LANES)]
                qv = iota16 + (k * LANES)
                lt2 = (v < bv) | ((v == bv) & (qv < bq))
                bv = jnp.where(lt2, v, bv)
                bq = jnp.where(lt2, qv, bq)
            mv = jnp.min(bv)
            _put(colmin_v, m2, mv)
            _put(colargq_v, m2, jnp.min(jnp.where(bv == mv, bq, BIGI)))

        def first_dirty(gq):
            bcol = BIGI * jnp.ones((LANES,), jnp.int32)
            for k in range(MP // LANES):
                qv = colargq_v[pl.ds(k * LANES, LANES)]
                cm = colmin_v[pl.ds(k * LANES, LANES)]
                msk = (qv == gq) & (cm < inf)
                bcol = jnp.minimum(bcol, jnp.where(msk, iota16 + (k * LANES), BIGI))
            return jnp.min(bcol)

        def body(_, carry):
            ccorr, bsum, asum = carry
            bv = colmin_v[pl.ds(0, LANES)]
            bq = colargq_v[pl.ds(0, LANES)]
            bm = iota16
            for k in range(1, MP // LANES):
                v = colmin_v[pl.ds(k * LANES, LANES)]
                qv = colargq_v[pl.ds(k * LANES, LANES)]
                mv_ = iota16 + (k * LANES)
                lt2 = (v < bv) | ((v == bv) & ((qv < bq) | ((qv == bq) & (mv_ < bm))))
                bv = jnp.where(lt2, v, bv)
                bq = jnp.where(lt2, qv, bq)
                bm = jnp.where(lt2, mv_, bm)
            gv = jnp.min(bv)
            c1 = bv == gv
            gq = jnp.min(jnp.where(c1, bq, BIGI))
            gm = jnp.min(jnp.where(c1 & (bq == gq), bm, BIGI))

            labm = _gat(lab_v, gm) - 1
            ccorr = (ccorr - _gat(logp_v, labm, gq)
                     + NOOBJ_W * _gat(logp_v, NCLS, gq))
            l1 = jnp.float32(0.0)
            for d in range(4):
                l1 = l1 + jnp.abs(_gat(pbt_v, d, gq) - _gat(tb_v, gm, d) * rs[d])
            bsum = bsum + l1
            dth = _gat(pbt_v, 4, gq) - _gat(tb_v, gm, 4)
            asum = asum + (1.0 - _cos_scalar(dth))

            _put(colmin_v, gm, inf)
            _put(qmask_v, gq, inf)

            def wbody(col):
                recompute_col(col)
                return first_dirty(gq)

            lax.while_loop(lambda col: col < BIGI, wbody, first_dirty(gq))
            return ccorr, bsum, asum

        z = jnp.float32(0.0)
        ccorr, bsum, asum = lax.fori_loop(0, M, body, (z, z, z))
        out16 = jnp.where(iota16 == 0, ccorr,
                          jnp.where(iota16 == 1, bsum,
                                    jnp.where(iota16 == 2, asum, 0.0)))
        outbuf_v[pl.ds(0, LANES)] = out16
        pltpu.sync_copy(outbuf_v, out_hbm.at[b])


def _sc_stage(cost, colmin, colargq, logp, pbt, tb, lab, size):
    mesh = plsc.VectorSubcoreMesh(core_axis_name="c", subcore_axis_name="s")
    f32 = jnp.float32
    fn = functools.partial(
        pl.kernel,
        mesh=mesh,
        compiler_params=pltpu.CompilerParams(needs_layout_passes=False),
        out_type=jax.ShapeDtypeStruct((B, 16), f32),
        scratch_types=[
            pltpu.VMEM((MP,), f32),
            pltpu.VMEM((MP,), jnp.int32),
            pltpu.VMEM((16, QP), f32),
            pltpu.VMEM((5, QP), f32),
            pltpu.VMEM((M, 5), f32),
            pltpu.VMEM((M,), jnp.int32),
            pltpu.VMEM((LANES,), jnp.int32),
            pltpu.VMEM((QP,), f32),
            pltpu.VMEM((QP,), f32),
            pltpu.VMEM((LANES,), f32),
        ],
    )(_sc_greedy)
    return fn(cost, colmin, colargq, logp, pbt, tb, lab, size)


@jax.jit
def kernel(pred_logits, pred_boxes, tgt_boxes, tgt_labels, tgt_size):
    f32 = jnp.float32
    lt = jnp.pad(jnp.swapaxes(pred_logits.astype(f32), 1, 2),
                 ((0, 0), (0, 0), (0, QP - Q)))
    pbt = jnp.pad(jnp.swapaxes(pred_boxes.astype(f32), 1, 2),
                  ((0, 0), (0, 0), (0, QP - Q)))
    tbp = jnp.pad(tgt_boxes.astype(f32), ((0, 0), (0, MP - M), (0, 0)))
    labp = jnp.pad(tgt_labels.astype(jnp.int32), ((0, 0), (0, MP - M)),
                   constant_values=NCLS + 1)[..., None]
    size32 = tgt_size.astype(jnp.int32)
    sizep = jnp.pad(size32, ((0, 0), (0, 14)))

    cost, colmin, colargq, logp, base = _tc_stage(
        size32[:, None, :], lt, pbt, tbp, labp)

    res = _sc_stage(cost, colmin.reshape(B, MP), colargq.reshape(B, MP),
                    logp, pbt, tgt_boxes.astype(f32),
                    tgt_labels.astype(jnp.int32), sizep)

    denom = NOOBJ_W * (Q - M) + 1.0 * M
    loss_cls = jnp.mean((base[:, 0, 0] + res[:, 0]) / denom)
    loss_bbox = jnp.mean(res[:, 1] / (M * 4)) * BBOX_W
    loss_ang = jnp.mean(res[:, 2] / M) * ANG_W
    return (loss_cls + loss_bbox + loss_ang, loss_cls, loss_bbox, loss_ang)


# speedup vs baseline: 49.7451x; 1.0504x over previous
import functools

import jax
import jax.numpy as jnp
from jax import lax
from jax.experimental import pallas as pl
from jax.experimental.pallas import tpu as pltpu
from jax.experimental.pallas import tpu_sc as plsc

NCLS = 15
CLS_W = 2.0
BBOX_W = 5.0
ANG_W = 2.0
NOOBJ_W = 0.1
B, Q, M = 4, 1000, 200
QP, MP = 1024, 256
LANES = 16
BIGI = 2 ** 30

_COS_C = [
    1.0, -0.5, 1.0 / 24, -1.0 / 720, 1.0 / 40320, -1.0 / 3628800,
    1.0 / 479001600, -1.0 / 87178291200,
]


def _cos_scalar(x):
    t = x * x
    r = jnp.float32(_COS_C[7])
    for k in range(6, -1, -1):
        r = r * t + jnp.float32(_COS_C[k])
    return r


def _tc_body(size_ref, lt_ref, pbt_ref, tb_ref, lab_ref,
             cost_ref, colmin_ref, colargq_ref, logp_ref, base_ref):
    lt = lt_ref[0]
    mx = jnp.max(lt, axis=0, keepdims=True)
    ex = jnp.exp(lt - mx)
    s = jnp.sum(ex, axis=0, keepdims=True)
    logp = lt - mx - jnp.log(s)
    logp_ref[0] = logp
    prob = ex / s

    hh = size_ref[0, 0, 0].astype(jnp.float32)
    ww = size_ref[0, 0, 1].astype(jnp.float32)
    tb = tb_ref[0]
    lab0 = lab_ref[0] - 1

    cls = jnp.zeros((MP, QP), jnp.float32)
    for c in range(NCLS):
        cls = cls + jnp.where(lab0 == c, prob[c:c + 1, :], 0.0)
    cost = cls * (-CLS_W)

    scales = (ww, hh, ww, hh)
    for d in range(4):
        nd = tb[:, d:d + 1] / scales[d]
        cost = cost + BBOX_W * jnp.abs(pbt_ref[0, d:d + 1, :] - nd)
    cost = cost + ANG_W * (1.0 - jnp.cos(pbt_ref[0, 4:5, :] - tb[:, 4:5]))

    qi = lax.broadcasted_iota(jnp.int32, (MP, QP), 1)
    mi = lax.broadcasted_iota(jnp.int32, (MP, QP), 0)
    cost = jnp.where((qi >= Q) | (mi >= M), jnp.inf, cost)
    cost_ref[0] = cost

    cmin = jnp.min(cost, axis=1, keepdims=True)
    colmin_ref[0] = cmin
    ismin = cost == cmin
    colargq_ref[0] = jnp.min(jnp.where(ismin, qi, QP), axis=1, keepdims=True)

    row15 = logp[NCLS:NCLS + 1, :]
    qrow = lax.broadcasted_iota(jnp.int32, (1, QP), 1)
    base_ref[0, 0, 0] = NOOBJ_W * jnp.sum(jnp.where(qrow < Q, -row15, 0.0))


def _tc_stage(size, lt, pbt, tbp, labp):
    f32 = jnp.float32
    out_shapes = (
        jax.ShapeDtypeStruct((B, MP, QP), f32),
        jax.ShapeDtypeStruct((B, MP, 1), f32),
        jax.ShapeDtypeStruct((B, MP, 1), jnp.int32),
        jax.ShapeDtypeStruct((B, 16, QP), f32),
        jax.ShapeDtypeStruct((B, 1, 1), f32),
    )
    grid = (B,)
    return pl.pallas_call(
        _tc_body,
        grid=grid,
        in_specs=[
            pl.BlockSpec((1, 1, 2), lambda b: (b, 0, 0), memory_space=pltpu.SMEM),
            pl.BlockSpec((1, 16, QP), lambda b: (b, 0, 0)),
            pl.BlockSpec((1, 5, QP), lambda b: (b, 0, 0)),
            pl.BlockSpec((1, MP, 5), lambda b: (b, 0, 0)),
            pl.BlockSpec((1, MP, 1), lambda b: (b, 0, 0)),
        ],
        out_specs=[
            pl.BlockSpec((1, MP, QP), lambda b: (b, 0, 0)),
            pl.BlockSpec((1, MP, 1), lambda b: (b, 0, 0)),
            pl.BlockSpec((1, MP, 1), lambda b: (b, 0, 0)),
            pl.BlockSpec((1, 16, QP), lambda b: (b, 0, 0)),
            pl.BlockSpec((1, 1, 1), lambda b: (b, 0, 0), memory_space=pltpu.SMEM),
        ],
        out_shape=out_shapes,
    )(size, lt, pbt, tbp, labp)


def _sc_greedy(cost_hbm, colmin_hbm, colargq_hbm, logp_hbm, pbt_hbm,
               tb_hbm, lab_hbm, size_hbm, out_hbm,
               colmin_v, colargq_v, logp_v, pbt_v, tb_v, lab_v, size_v,
               qmask_v, rowbuf_v, outbuf_v):
    info = plsc.get_sparse_core_info()
    ns = info.num_subcores
    wid = lax.axis_index("c") * ns + lax.axis_index("s")

    iota16 = lax.broadcasted_iota(jnp.int32, (LANES,), 0)
    lane0 = iota16 == 0

    def _gat(ref, *idx):
        idxs = [jnp.broadcast_to(i, (LANES,)).astype(jnp.int32) for i in idx]
        return plsc.load_gather(ref, idxs)[0]

    def _put(ref, i, val):
        ii = jnp.broadcast_to(i, (LANES,)).astype(jnp.int32)
        plsc.store_scatter(ref, [ii], jnp.broadcast_to(val, (LANES,)),
                           mask=lane0)

    @pl.when(wid < B)
    def _work():
        b = wid
        pltpu.sync_copy(colmin_hbm.at[b], colmin_v)
        pltpu.sync_copy(colargq_hbm.at[b], colargq_v)
        pltpu.sync_copy(logp_hbm.at[b], logp_v)
        pltpu.sync_copy(pbt_hbm.at[b], pbt_v)
        pltpu.sync_copy(tb_hbm.at[b], tb_v)
        pltpu.sync_copy(lab_hbm.at[b], lab_v)
        pltpu.sync_copy(size_hbm.at[b], size_v)

        zeros16 = jnp.zeros((LANES,), jnp.float32)
        for k in range(QP // LANES):
            qmask_v[pl.ds(k * LANES, LANES)] = zeros16

        sizes = size_v[pl.ds(0, LANES)]
        rcp = 1.0 / sizes.astype(jnp.float32)
        rw = rcp[1]
        rh = rcp[0]
        rs = (rw, rh, rw, rh)
        inf = jnp.float32(jnp.inf)

        def recompute_col(m2):
            pltpu.sync_copy(cost_hbm.at[b, m2], rowbuf_v)
            bv = rowbuf_v[pl.ds(0, LANES)] + qmask_v[pl.ds(0, LANES)]
            bq = iota16
            for k in range(1, QP // LANES):
                v = rowbuf_v[pl.ds(k * LANES, LANES)] + qmask_v[pl.ds(k * LANES, LANES)]
                qv = iota16 + (k * LANES)
                lt2 = (v < bv) | ((v == bv) & (qv < bq))
                bv = jnp.where(lt2, v, bv)
                bq = jnp.where(lt2, qv, bq)
            mv = jnp.min(bv)
            _put(colmin_v, m2, mv)
            _put(colargq_v, m2, jnp.min(jnp.where(bv == mv, bq, BIGI)))

        def scan_min():
            bv = colmin_v[pl.ds(0, LANES)]
            bq = colargq_v[pl.ds(0, LANES)]
            bm = iota16
            for k in range(1, MP // LANES):
                v = colmin_v[pl.ds(k * LANES, LANES)]
                qv = colargq_v[pl.ds(k * LANES, LANES)]
                mv_ = iota16 + (k * LANES)
                lt2 = (v < bv) | ((v == bv) & ((qv < bq) | ((qv == bq) & (mv_ < bm))))
                bv = jnp.where(lt2, v, bv)
                bq = jnp.where(lt2, qv, bq)
                bm = jnp.where(lt2, mv_, bm)
            gv = jnp.min(bv)
            c1 = bv == gv
            gq = jnp.min(jnp.where(c1, bq, BIGI))
            gm = jnp.min(jnp.where(c1 & (bq == gq), bm, BIGI))
            return gv, gq, gm

        def body(_, carry):
            ccorr, bsum, asum = carry
            def wcond(st):
                _, gq0, _ = st
                return _gat(qmask_v, gq0) != 0.0

            def wbody(st):
                _, _, gm0 = st
                recompute_col(gm0)
                return scan_min()

            gv, gq, gm = lax.while_loop(wcond, wbody, scan_min())

            labm = _gat(lab_v, gm) - 1
            ccorr = (ccorr - _gat(logp_v, labm, gq)
                     + NOOBJ_W * _gat(logp_v, NCLS, gq))
            l1 = jnp.float32(0.0)
            for d in range(4):
                l1 = l1 + jnp.abs(_gat(pbt_v, d, gq) - _gat(tb_v, gm, d) * rs[d])
            bsum = bsum + l1
            dth = _gat(pbt_v, 4, gq) - _gat(tb_v, gm, 4)
            asum = asum + (1.0 - _cos_scalar(dth))

            _put(colmin_v, gm, inf)
            _put(qmask_v, gq, inf)
            return ccorr, bsum, asum

        z = jnp.float32(0.0)
        ccorr, bsum, asum = lax.fori_loop(0, M, body, (z, z, z))
        out16 = jnp.where(iota16 == 0, ccorr,
                          jnp.where(iota16 == 1, bsum,
                                    jnp.where(iota16 == 2, asum, 0.0)))
        outbuf_v[pl.ds(0, LANES)] = out16
        pltpu.sync_copy(outbuf_v, out_hbm.at[b])


def _sc_stage(cost, colmin, colargq, logp, pbt, tb, lab, size):
    mesh = plsc.VectorSubcoreMesh(core_axis_name="c", subcore_axis_name="s")
    f32 = jnp.float32
    fn = functools.partial(
        pl.kernel,
        mesh=mesh,
        compiler_params=pltpu.CompilerParams(needs_layout_passes=False),
        out_type=jax.ShapeDtypeStruct((B, 16), f32),
        scratch_types=[
            pltpu.VMEM((MP,), f32),
            pltpu.VMEM((MP,), jnp.int32),
            pltpu.VMEM((16, QP), f32),
            pltpu.VMEM((5, QP), f32),
            pltpu.VMEM((M, 5), f32),
            pltpu.VMEM((M,), jnp.int32),
            pltpu.VMEM((LANES,), jnp.int32),
            pltpu.VMEM((QP,), f32),
            pltpu.VMEM((QP,), f32),
            pltpu.VMEM((LANES,), f32),
        ],
    )(_sc_greedy)
    return fn(cost, colmin, colargq, logp, pbt, tb, lab, size)


@jax.jit
def kernel(pred_logits, pred_boxes, tgt_boxes, tgt_labels, tgt_size):
    f32 = jnp.float32
    lt = jnp.pad(jnp.swapaxes(pred_logits.astype(f32), 1, 2),
                 ((0, 0), (0, 0), (0, QP - Q)))
    pbt = jnp.pad(jnp.swapaxes(pred_boxes.astype(f32), 1, 2),
                  ((0, 0), (0, 0), (0, QP - Q)))
    tbp = jnp.pad(tgt_boxes.astype(f32), ((0, 0), (0, MP - M), (0, 0)))
    labp = jnp.pad(tgt_labels.astype(jnp.int32), ((0, 0), (0, MP - M)),
                   constant_values=NCLS + 1)[..., None]
    size32 = tgt_size.astype(jnp.int32)
    sizep = jnp.pad(size32, ((0, 0), (0, 14)))

    cost, colmin, colargq, logp, base = _tc_stage(
        size32[:, None, :], lt, pbt, tbp, labp)

    res = _sc_stage(cost, colmin.reshape(B, MP), colargq.reshape(B, MP),
                    logp, pbt, tgt_boxes.astype(f32),
                    tgt_labels.astype(jnp.int32), sizep)

    denom = NOOBJ_W * (Q - M) + 1.0 * M
    loss_cls = jnp.mean((base[:, 0, 0] + res[:, 0]) / denom)
    loss_bbox = jnp.mean(res[:, 1] / (M * 4)) * BBOX_W
    loss_ang = jnp.mean(res[:, 2] / M) * ANG_W
    return (loss_cls + loss_bbox + loss_ang, loss_cls, loss_bbox, loss_ang)


# speedup vs baseline: 52.6035x; 1.0575x over previous
import functools

import jax
import jax.numpy as jnp
from jax import lax
from jax.experimental import pallas as pl
from jax.experimental.pallas import tpu as pltpu
from jax.experimental.pallas import tpu_sc as plsc

NCLS = 15
CLS_W = 2.0
BBOX_W = 5.0
ANG_W = 2.0
NOOBJ_W = 0.1
B, Q, M = 4, 1000, 200
QP, MP = 1024, 256
NPP = NCLS + 1 + 5
LANES = 16
BIGI = 2 ** 30

_COS_C = [
    1.0, -0.5, 1.0 / 24, -1.0 / 720, 1.0 / 40320, -1.0 / 3628800,
    1.0 / 479001600, -1.0 / 87178291200,
]


def _cos_scalar(x):
    t = x * x
    r = jnp.float32(_COS_C[7])
    for k in range(6, -1, -1):
        r = r * t + jnp.float32(_COS_C[k])
    return r


def _tc_body(size_ref, lt_ref, pbt_ref, tb_ref, lab_ref,
             cost_ref, colmin_ref, colargq_ref, pp_ref, base_ref):
    lt = lt_ref[0]
    mx = jnp.max(lt, axis=0, keepdims=True)
    ex = jnp.exp(lt - mx)
    s = jnp.sum(ex, axis=0, keepdims=True)
    logp = lt - mx - jnp.log(s)
    pp_ref[0, :NCLS + 1, :] = logp
    pp_ref[0, NCLS + 1:, :] = pbt_ref[0]
    prob = ex / s

    hh = size_ref[0, 0, 0].astype(jnp.float32)
    ww = size_ref[0, 0, 1].astype(jnp.float32)
    tb = tb_ref[0]
    lab0 = lab_ref[0] - 1

    sel = [prob[c:c + 1, :] for c in range(16)]
    for bit in (1, 2, 4, 8):
        cond = (lab0 & bit) != 0
        sel = [jnp.where(cond, sel[i + 1], sel[i])
               for i in range(0, len(sel), 2)]
    cls_cost = sel[0] * (-CLS_W)

    di = lax.broadcasted_iota(jnp.int32, (1, 5), 1)
    scale5 = jnp.where((di == 0) | (di == 2), ww,
                       jnp.where(di == 4, 1.0, hh))
    tbn = tb / scale5
    l1 = jnp.abs(pbt_ref[0, 0:1, :] - tbn[:, 0:1])
    for d in range(1, 4):
        l1 = l1 + jnp.abs(pbt_ref[0, d:d + 1, :] - tbn[:, d:d + 1])
    pth = pbt_ref[0, 4:5, :]
    tth = tb[:, 4:5]
    ang = 1.0 - (jnp.cos(pth) * jnp.cos(tth) + jnp.sin(pth) * jnp.sin(tth))
    cost = cls_cost + l1 * BBOX_W + ang * ANG_W

    qi = lax.broadcasted_iota(jnp.int32, (MP, QP), 1)
    mi = lax.broadcasted_iota(jnp.int32, (MP, QP), 0)
    cost = jnp.where((qi >= Q) | (mi >= M), jnp.inf, cost)
    cost_ref[0] = cost

    cmin = jnp.min(cost, axis=1, keepdims=True)
    colmin_ref[0] = cmin
    ismin = cost == cmin
    colargq_ref[0] = jnp.min(jnp.where(ismin, qi, QP), axis=1, keepdims=True)

    row15 = logp[NCLS:NCLS + 1, :]
    qrow = lax.broadcasted_iota(jnp.int32, (1, QP), 1)
    base_ref[0, 0, 0] = NOOBJ_W * jnp.sum(jnp.where(qrow < Q, -row15, 0.0))


def _tc_stage(size, lt, pbt, tbp, labp):
    f32 = jnp.float32
    out_shapes = (
        jax.ShapeDtypeStruct((B, MP, QP), f32),
        jax.ShapeDtypeStruct((B, MP, 1), f32),
        jax.ShapeDtypeStruct((B, MP, 1), jnp.int32),
        jax.ShapeDtypeStruct((B, NPP, QP), f32),
        jax.ShapeDtypeStruct((B, 1, 1), f32),
    )
    grid = (B,)
    return pl.pallas_call(
        _tc_body,
        grid=grid,
        in_specs=[
            pl.BlockSpec((1, 1, 2), lambda b: (b, 0, 0), memory_space=pltpu.SMEM),
            pl.BlockSpec((1, 16, QP), lambda b: (b, 0, 0)),
            pl.BlockSpec((1, 5, QP), lambda b: (b, 0, 0)),
            pl.BlockSpec((1, MP, 5), lambda b: (b, 0, 0)),
            pl.BlockSpec((1, MP, 1), lambda b: (b, 0, 0)),
        ],
        out_specs=[
            pl.BlockSpec((1, MP, QP), lambda b: (b, 0, 0)),
            pl.BlockSpec((1, MP, 1), lambda b: (b, 0, 0)),
            pl.BlockSpec((1, MP, 1), lambda b: (b, 0, 0)),
            pl.BlockSpec((1, NPP, QP), lambda b: (b, 0, 0)),
            pl.BlockSpec((1, 1, 1), lambda b: (b, 0, 0), memory_space=pltpu.SMEM),
        ],
        out_shape=out_shapes,
    )(size, lt, pbt, tbp, labp)


def _sc_greedy(cost_hbm, colmin_hbm, colargq_hbm, pp_hbm, tg_hbm, size_hbm,
               out_hbm,
               colmin_v, colargq_v, pp_v, tg_v, size_v,
               qmask_v, rowbuf_v, outbuf_v, summary_v, cost_sh):
    info = plsc.get_sparse_core_info()
    ns = info.num_subcores
    wid = lax.axis_index("c") * ns + lax.axis_index("s")

    iota16 = lax.broadcasted_iota(jnp.int32, (LANES,), 0)
    lane0 = iota16 == 0

    def _gat(ref, *idx):
        idxs = [jnp.broadcast_to(i, (LANES,)).astype(jnp.int32) for i in idx]
        return plsc.load_gather(ref, idxs)[0]

    def _put(ref, i, val):
        ii = jnp.broadcast_to(i, (LANES,)).astype(jnp.int32)
        plsc.store_scatter(ref, [ii], jnp.broadcast_to(val, (LANES,)),
                           mask=lane0)

    @pl.when(wid < B)
    def _work():
        b = wid
        pltpu.sync_copy(cost_hbm.at[b], cost_sh.at[b])
        pltpu.sync_copy(colmin_hbm.at[b], colmin_v)
        pltpu.sync_copy(colargq_hbm.at[b], colargq_v)
        pltpu.sync_copy(pp_hbm.at[b], pp_v)
        pltpu.sync_copy(tg_hbm.at[b], tg_v)
        pltpu.sync_copy(size_hbm.at[b], size_v)

        zeros16 = jnp.zeros((LANES,), jnp.float32)
        for k in range(QP // LANES):
            qmask_v[pl.ds(k * LANES, LANES)] = zeros16
        for k in range(MP // LANES):
            _put(summary_v, k, jnp.min(colmin_v[pl.ds(k * LANES, LANES)]))

        sizes = size_v[pl.ds(0, LANES)]
        rcp = 1.0 / sizes.astype(jnp.float32)
        rw = rcp[1]
        rh = rcp[0]
        rs = (rw, rh, rw, rh)
        inf = jnp.float32(jnp.inf)

        def upd_summary(m):
            k = lax.shift_right_logical(m, 4)
            _put(summary_v, k, jnp.min(colmin_v[pl.ds(k * LANES, LANES)]))

        def recompute_col(m2):
            pltpu.sync_copy(cost_sh.at[b, m2], rowbuf_v)
            bv = rowbuf_v[pl.ds(0, LANES)] + qmask_v[pl.ds(0, LANES)]
            bq = iota16
            for k in range(1, QP // LANES):
                v = rowbuf_v[pl.ds(k * LANES, LANES)] + qmask_v[pl.ds(k * LANES, LANES)]
                qv = iota16 + (k * LANES)
                lt2 = (v < bv) | ((v == bv) & (qv < bq))
                bv = jnp.where(lt2, v, bv)
                bq = jnp.where(lt2, qv, bq)
            mv = jnp.min(bv)
            _put(colmin_v, m2, mv)
            _put(colargq_v, m2, jnp.min(jnp.where(bv == mv, bq, BIGI)))
            upd_summary(m2)

        def chunk_best(k):
            off = k * LANES
            v = colmin_v[pl.ds(off, LANES)]
            qv = colargq_v[pl.ds(off, LANES)]
            gv = jnp.min(v)
            c1 = v == gv
            gq = jnp.min(jnp.where(c1, qv, BIGI))
            gm = off + jnp.min(jnp.where(c1 & (qv == gq), iota16, BIGI))
            return gv, gq, gm

        def scan_min():
            sv = summary_v[pl.ds(0, LANES)]
            cand = sv == jnp.min(sv)

            def ccond(st):
                mask = st[0]
                return jnp.sum(mask.astype(jnp.int32)) > 0

            def cbody(st):
                mask, bv, bq, bm = st
                k = jnp.min(jnp.where(mask, iota16, BIGI))
                v2, q2, m2 = chunk_best(k)
                bt = (v2 < bv) | ((v2 == bv) & ((q2 < bq) | ((q2 == bq) & (m2 < bm))))
                return (mask & (iota16 != k), jnp.where(bt, v2, bv),
                        jnp.where(bt, q2, bq), jnp.where(bt, m2, bm))

            _, gv, gq, gm = lax.while_loop(
                ccond, cbody,
                (cand, jnp.float32(jnp.inf), jnp.int32(BIGI), jnp.int32(BIGI)))
            return gv, gq, gm

        def body(_, carry):
            ccorr, bsum, asum = carry
            def wcond(st):
                _, gq0, _ = st
                return _gat(qmask_v, gq0) != 0.0

            def wbody(st):
                _, _, gm0 = st
                recompute_col(gm0)
                return scan_min()

            gv, gq, gm = lax.while_loop(wcond, wbody, scan_min())

            g1 = plsc.load_gather(tg_v, [gm * 8 + jnp.minimum(iota16, 5)])
            labm = g1[5].astype(jnp.int32) - 1
            rowsel = jnp.where(iota16 == 0, labm,
                               jnp.where(iota16 == 1, NCLS,
                                         jnp.minimum(iota16 + 14, NPP - 1)))
            g2 = plsc.load_gather(pp_v, [rowsel * QP + gq])
            ccorr = ccorr - g2[0] + NOOBJ_W * g2[1]
            l1 = jnp.abs(g2[2] - g1[0] * rs[0])
            for d in range(1, 4):
                l1 = l1 + jnp.abs(g2[2 + d] - g1[d] * rs[d])
            bsum = bsum + l1
            dth = g2[6] - g1[4]
            asum = asum + (1.0 - _cos_scalar(dth))

            _put(colmin_v, gm, inf)
            _put(qmask_v, gq, inf)
            upd_summary(gm)
            return ccorr, bsum, asum

        z = jnp.float32(0.0)
        ccorr, bsum, asum = lax.fori_loop(0, M, body, (z, z, z))
        out16 = jnp.where(iota16 == 0, ccorr,
                          jnp.where(iota16 == 1, bsum,
                                    jnp.where(iota16 == 2, asum, 0.0)))
        outbuf_v[pl.ds(0, LANES)] = out16
        pltpu.sync_copy(outbuf_v, out_hbm.at[b])


def _sc_stage(cost, colmin, colargq, pp, tg, size):
    mesh = plsc.VectorSubcoreMesh(core_axis_name="c", subcore_axis_name="s")
    f32 = jnp.float32
    fn = functools.partial(
        pl.kernel,
        mesh=mesh,
        compiler_params=pltpu.CompilerParams(needs_layout_passes=False),
        out_type=jax.ShapeDtypeStruct((B, 16), f32),
        scratch_types=[
            pltpu.VMEM((MP,), f32),
            pltpu.VMEM((MP,), jnp.int32),
            pltpu.VMEM((NPP * QP,), f32),
            pltpu.VMEM((M * 8,), f32),
            pltpu.VMEM((LANES,), jnp.int32),
            pltpu.VMEM((QP,), f32),
            pltpu.VMEM((QP,), f32),
            pltpu.VMEM((LANES,), f32),
            pltpu.VMEM((LANES,), f32),
            pltpu.VMEM_SHARED((B, MP, QP), f32),
        ],
    )(_sc_greedy)
    return fn(cost, colmin, colargq, pp, tg, size)


@jax.jit
def kernel(pred_logits, pred_boxes, tgt_boxes, tgt_labels, tgt_size):
    f32 = jnp.float32
    lt = jnp.pad(jnp.swapaxes(pred_logits.astype(f32), 1, 2),
                 ((0, 0), (0, 0), (0, QP - Q)))
    pbt = jnp.pad(jnp.swapaxes(pred_boxes.astype(f32), 1, 2),
                  ((0, 0), (0, 0), (0, QP - Q)))
    tbp = jnp.pad(tgt_boxes.astype(f32), ((0, 0), (0, MP - M), (0, 0)))
    labp = jnp.pad(tgt_labels.astype(jnp.int32), ((0, 0), (0, MP - M)),
                   constant_values=NCLS + 1)[..., None]
    size32 = tgt_size.astype(jnp.int32)
    sizep = jnp.pad(size32, ((0, 0), (0, 14)))
    tg = jnp.concatenate(
        [tgt_boxes.astype(f32), tgt_labels.astype(f32)[..., None],
         jnp.zeros((B, M, 2), f32)], axis=-1).reshape(B, M * 8)

    cost, colmin, colargq, pp, base = _tc_stage(
        size32[:, None, :], lt, pbt, tbp, labp)

    res = _sc_stage(cost, colmin.reshape(B, MP), colargq.reshape(B, MP),
                    pp.reshape(B, NPP * QP), tg, sizep)

    denom = NOOBJ_W * (Q - M) + 1.0 * M
    loss_cls = jnp.mean((base[:, 0, 0] + res[:, 0]) / denom)
    loss_bbox = jnp.mean(res[:, 1] / (M * 4)) * BBOX_W
    loss_ang = jnp.mean(res[:, 2] / M) * ANG_W
    return (loss_cls + loss_bbox + loss_ang, loss_cls, loss_bbox, loss_ang)


# speedup vs baseline: 62.3992x; 1.1862x over previous
import functools

import jax
import jax.numpy as jnp
from jax import lax
from jax.experimental import pallas as pl
from jax.experimental.pallas import tpu as pltpu
from jax.experimental.pallas import tpu_sc as plsc

NCLS = 15
CLS_W = 2.0
BBOX_W = 5.0
ANG_W = 2.0
NOOBJ_W = 0.1
B, Q, M = 4, 1000, 200
QP, MP = 1024, 256
NPP = NCLS + 1 + 5
LANES = 16
BIGI = 2 ** 30

_COS_C = [
    1.0, -0.5, 1.0 / 24, -1.0 / 720, 1.0 / 40320, -1.0 / 3628800,
    1.0 / 479001600, -1.0 / 87178291200,
]


def _cos_scalar(x):
    t = x * x
    r = jnp.float32(_COS_C[7])
    for k in range(6, -1, -1):
        r = r * t + jnp.float32(_COS_C[k])
    return r


def _tc_body(size_ref, lt_ref, pbt_ref, tb_ref, lab_ref,
             cost_ref, colmin_ref, colargq_ref, pp_ref, base_ref):
    lt = lt_ref[0]
    mx = jnp.max(lt, axis=0, keepdims=True)
    ex = jnp.exp(lt - mx)
    s = jnp.sum(ex, axis=0, keepdims=True)
    logp = lt - mx - jnp.log(s)
    pp_ref[0, :NCLS + 1, :] = logp
    pp_ref[0, NCLS + 1:, :] = pbt_ref[0]
    prob = ex / s

    hh = size_ref[0, 0, 0].astype(jnp.float32)
    ww = size_ref[0, 0, 1].astype(jnp.float32)
    tb = tb_ref[0]
    lab0 = lab_ref[0] - 1

    sel = [prob[c:c + 1, :] for c in range(16)]
    for bit in (1, 2, 4, 8):
        cond = (lab0 & bit) != 0
        sel = [jnp.where(cond, sel[i + 1], sel[i])
               for i in range(0, len(sel), 2)]
    cls_cost = sel[0] * (-CLS_W)

    di = lax.broadcasted_iota(jnp.int32, (1, 5), 1)
    scale5 = jnp.where((di == 0) | (di == 2), ww,
                       jnp.where(di == 4, 1.0, hh))
    tbn = tb / scale5
    l1 = jnp.abs(pbt_ref[0, 0:1, :] - tbn[:, 0:1])
    for d in range(1, 4):
        l1 = l1 + jnp.abs(pbt_ref[0, d:d + 1, :] - tbn[:, d:d + 1])
    pth = pbt_ref[0, 4:5, :]
    tth = tb[:, 4:5]
    ang = 1.0 - (jnp.cos(pth) * jnp.cos(tth) + jnp.sin(pth) * jnp.sin(tth))
    cost = cls_cost + l1 * BBOX_W + ang * ANG_W

    qi = lax.broadcasted_iota(jnp.int32, (MP, QP), 1)
    mi = lax.broadcasted_iota(jnp.int32, (MP, QP), 0)
    cost = jnp.where((qi >= Q) | (mi >= M), jnp.inf, cost)
    cost_ref[0] = cost

    cmin = jnp.min(cost, axis=1, keepdims=True)
    colmin_ref[0] = cmin
    ismin = cost == cmin
    colargq_ref[0] = jnp.min(jnp.where(ismin, qi, QP), axis=1, keepdims=True)

    row15 = logp[NCLS:NCLS + 1, :]
    qrow = lax.broadcasted_iota(jnp.int32, (1, QP), 1)
    base_ref[0, 0, 0] = NOOBJ_W * jnp.sum(jnp.where(qrow < Q, -row15, 0.0))


def _tc_stage(size, lt, pbt, tbp, labp):
    f32 = jnp.float32
    out_shapes = (
        jax.ShapeDtypeStruct((B, MP, QP), f32),
        jax.ShapeDtypeStruct((B, MP, 1), f32),
        jax.ShapeDtypeStruct((B, MP, 1), jnp.int32),
        jax.ShapeDtypeStruct((B, NPP, QP), f32),
        jax.ShapeDtypeStruct((B, 1, 1), f32),
    )
    grid = (B,)
    return pl.pallas_call(
        _tc_body,
        grid=grid,
        in_specs=[
            pl.BlockSpec((1, 1, 2), lambda b: (b, 0, 0), memory_space=pltpu.SMEM),
            pl.BlockSpec((1, 16, QP), lambda b: (b, 0, 0)),
            pl.BlockSpec((1, 5, QP), lambda b: (b, 0, 0)),
            pl.BlockSpec((1, MP, 5), lambda b: (b, 0, 0)),
            pl.BlockSpec((1, MP, 1), lambda b: (b, 0, 0)),
        ],
        out_specs=[
            pl.BlockSpec((1, MP, QP), lambda b: (b, 0, 0)),
            pl.BlockSpec((1, MP, 1), lambda b: (b, 0, 0)),
            pl.BlockSpec((1, MP, 1), lambda b: (b, 0, 0)),
            pl.BlockSpec((1, NPP, QP), lambda b: (b, 0, 0)),
            pl.BlockSpec((1, 1, 1), lambda b: (b, 0, 0), memory_space=pltpu.SMEM),
        ],
        out_shape=out_shapes,
    )(size, lt, pbt, tbp, labp)


def _sc_greedy(cost_hbm, colmin_hbm, colargq_hbm, pp_hbm, tg_hbm, size_hbm,
               out_hbm,
               colmin_v, colargq_v, pp_v, tg_v, size_v,
               qmask_v, rowbuf_v, outbuf_v, summary_v, mq_v, mm_v, cost_sh):
    info = plsc.get_sparse_core_info()
    ns = info.num_subcores
    wid = lax.axis_index("c") * ns + lax.axis_index("s")

    iota16 = lax.broadcasted_iota(jnp.int32, (LANES,), 0)
    lane0 = iota16 == 0

    def _gat(ref, *idx):
        idxs = [jnp.broadcast_to(i, (LANES,)).astype(jnp.int32) for i in idx]
        return plsc.load_gather(ref, idxs)[0]

    def _gatv(ref, idx16):
        return plsc.load_gather(ref, [idx16])

    def _put(ref, i, val):
        ii = jnp.broadcast_to(i, (LANES,)).astype(jnp.int32)
        plsc.store_scatter(ref, [ii], jnp.broadcast_to(val, (LANES,)),
                           mask=lane0)

    @pl.when(wid < B)
    def _work():
        b = wid
        with jax.named_scope("sc_stage_in"):
            pltpu.sync_copy(cost_hbm.at[b], cost_sh.at[b])
            pltpu.sync_copy(colmin_hbm.at[b], colmin_v)
            pltpu.sync_copy(colargq_hbm.at[b], colargq_v)
            pltpu.sync_copy(pp_hbm.at[b], pp_v)
            pltpu.sync_copy(tg_hbm.at[b], tg_v)
            pltpu.sync_copy(size_hbm.at[b], size_v)

        zeros16 = jnp.zeros((LANES,), jnp.float32)
        for k in range(QP // LANES):
            qmask_v[pl.ds(k * LANES, LANES)] = zeros16
        for k in range(MP // LANES):
            _put(summary_v, k, jnp.min(colmin_v[pl.ds(k * LANES, LANES)]))
        mq_v[pl.ds(M - 8, LANES)] = jnp.zeros((LANES,), jnp.int32)
        mm_v[pl.ds(M - 8, LANES)] = jnp.zeros((LANES,), jnp.int32)

        sizes = size_v[pl.ds(0, LANES)]
        rcp = 1.0 / sizes.astype(jnp.float32)
        rw = rcp[1]
        rh = rcp[0]
        rs = (rw, rh, rw, rh)
        inf = jnp.float32(jnp.inf)

        def upd_summary(m):
            k = lax.shift_right_logical(m, 4)
            _put(summary_v, k, jnp.min(colmin_v[pl.ds(k * LANES, LANES)]))

        def recompute_col(m2):
            pltpu.sync_copy(cost_sh.at[b, m2], rowbuf_v)
            bv = rowbuf_v[pl.ds(0, LANES)] + qmask_v[pl.ds(0, LANES)]
            bq = iota16
            for k in range(1, QP // LANES):
                v = rowbuf_v[pl.ds(k * LANES, LANES)] + qmask_v[pl.ds(k * LANES, LANES)]
                qv = iota16 + (k * LANES)
                lt2 = (v < bv) | ((v == bv) & (qv < bq))
                bv = jnp.where(lt2, v, bv)
                bq = jnp.where(lt2, qv, bq)
            mv = jnp.min(bv)
            _put(colmin_v, m2, mv)
            _put(colargq_v, m2, jnp.min(jnp.where(bv == mv, bq, BIGI)))
            upd_summary(m2)

        def full_scan():
            bv = colmin_v[pl.ds(0, LANES)]
            bq = colargq_v[pl.ds(0, LANES)]
            bm = iota16
            for k in range(1, MP // LANES):
                v = colmin_v[pl.ds(k * LANES, LANES)]
                qv = colargq_v[pl.ds(k * LANES, LANES)]
                mv_ = iota16 + (k * LANES)
                lt2 = (v < bv) | ((v == bv) & ((qv < bq) | ((qv == bq) & (mv_ < bm))))
                bv = jnp.where(lt2, v, bv)
                bq = jnp.where(lt2, qv, bq)
                bm = jnp.where(lt2, mv_, bm)
            gv = jnp.min(bv)
            c1 = bv == gv
            gq = jnp.min(jnp.where(c1, bq, BIGI))
            gm = jnp.min(jnp.where(c1 & (bq == gq), bm, BIGI))
            return gv, gq, gm

        def scan_min():
            sv = summary_v[pl.ds(0, LANES)]
            sk, skidx = plsc.sort_key_val(sv, iota16)
            k0 = skidx[0]
            off = k0 * LANES
            v = colmin_v[pl.ds(off, LANES)]
            qv = colargq_v[pl.ds(off, LANES)]
            ck, cp = plsc.sort_key_val(v, qv * LANES + iota16)
            p0 = cp[0]
            tie = (sk[1] == sk[0]) | (ck[1] == ck[0])
            return lax.cond(
                tie, full_scan,
                lambda: (ck[0], lax.shift_right_logical(p0, 4),
                         off + (p0 & (LANES - 1))))

        def body(it, carry):
            def wcond(st):
                _, gq0, _ = st
                return _gat(qmask_v, gq0) != 0.0

            def wbody(st):
                _, _, gm0 = st
                recompute_col(gm0)
                return scan_min()

            gv, gq, gm = lax.while_loop(wcond, wbody, scan_min())

            _put(mq_v, it, gq)
            _put(mm_v, it, gm)

            _put(colmin_v, gm, inf)
            _put(qmask_v, gq, inf)
            upd_summary(gm)
            return carry

        with jax.named_scope("sc_match"):
            lax.fori_loop(0, M, body, 0)

        z16 = jnp.zeros((LANES,), jnp.float32)
        ccorr_v = z16
        bsum_v = z16
        asum_v = z16
        for g in range(MP // LANES):
            base_i = g * LANES
            if base_i >= M:
                break
            q16 = mq_v[pl.ds(base_i, LANES)]
            m16 = mm_v[pl.ds(base_i, LANES)]
            labm16 = _gatv(tg_v, m16 * 8 + 5).astype(jnp.int32) - 1
            lp = _gatv(pp_v, labm16 * QP + q16)
            lp15 = _gatv(pp_v, NCLS * QP + q16)
            t = [_gatv(tg_v, m16 * 8 + d) for d in range(5)]
            p = [_gatv(pp_v, (NCLS + 1 + d) * QP + q16) for d in range(5)]
            cc = -lp + NOOBJ_W * lp15
            l1 = jnp.abs(p[0] - t[0] * rs[0])
            for d in range(1, 4):
                l1 = l1 + jnp.abs(p[d] - t[d] * rs[d])
            dth = p[4] - t[4]
            av = 1.0 - _cos_scalar(dth)
            if base_i + LANES > M:
                valid = iota16 < (M - base_i)
                cc = jnp.where(valid, cc, 0.0)
                l1 = jnp.where(valid, l1, 0.0)
                av = jnp.where(valid, av, 0.0)
            ccorr_v = ccorr_v + cc
            bsum_v = bsum_v + l1
            asum_v = asum_v + av
        ccorr = jnp.sum(ccorr_v)
        bsum = jnp.sum(bsum_v)
        asum = jnp.sum(asum_v)
        out16 = jnp.where(iota16 == 0, ccorr,
                          jnp.where(iota16 == 1, bsum,
                                    jnp.where(iota16 == 2, asum, 0.0)))
        outbuf_v[pl.ds(0, LANES)] = out16
        pltpu.sync_copy(outbuf_v, out_hbm.at[b])


def _sc_stage(cost, colmin, colargq, pp, tg, size):
    mesh = plsc.VectorSubcoreMesh(core_axis_name="c", subcore_axis_name="s")
    f32 = jnp.float32
    fn = functools.partial(
        pl.kernel,
        mesh=mesh,
        compiler_params=pltpu.CompilerParams(needs_layout_passes=False),
        out_type=jax.ShapeDtypeStruct((B, 16), f32),
        scratch_types=[
            pltpu.VMEM((MP,), f32),
            pltpu.VMEM((MP,), jnp.int32),
            pltpu.VMEM((NPP * QP,), f32),
            pltpu.VMEM((M * 8,), f32),
            pltpu.VMEM((LANES,), jnp.int32),
            pltpu.VMEM((QP,), f32),
            pltpu.VMEM((QP,), f32),
            pltpu.VMEM((LANES,), f32),
            pltpu.VMEM((LANES,), f32),
            pltpu.VMEM((MP,), jnp.int32),
            pltpu.VMEM((MP,), jnp.int32),
            pltpu.VMEM_SHARED((B, MP, QP), f32),
        ],
    )(_sc_greedy)
    return fn(cost, colmin, colargq, pp, tg, size)


@jax.jit
def kernel(pred_logits, pred_boxes, tgt_boxes, tgt_labels, tgt_size):
    f32 = jnp.float32
    lt = jnp.pad(jnp.swapaxes(pred_logits.astype(f32), 1, 2),
                 ((0, 0), (0, 0), (0, QP - Q)))
    pbt = jnp.pad(jnp.swapaxes(pred_boxes.astype(f32), 1, 2),
                  ((0, 0), (0, 0), (0, QP - Q)))
    tbp = jnp.pad(tgt_boxes.astype(f32), ((0, 0), (0, MP - M), (0, 0)))
    labp = jnp.pad(tgt_labels.astype(jnp.int32), ((0, 0), (0, MP - M)),
                   constant_values=NCLS + 1)[..., None]
    size32 = tgt_size.astype(jnp.int32)
    sizep = jnp.pad(size32, ((0, 0), (0, 14)))
    tg = jnp.concatenate(
        [tgt_boxes.astype(f32), tgt_labels.astype(f32)[..., None],
         jnp.zeros((B, M, 2), f32)], axis=-1).reshape(B, M * 8)

    cost, colmin, colargq, pp, base = _tc_stage(
        size32[:, None, :], lt, pbt, tbp, labp)

    res = _sc_stage(cost, colmin.reshape(B, MP), colargq.reshape(B, MP),
                    pp.reshape(B, NPP * QP), tg, sizep)

    denom = NOOBJ_W * (Q - M) + 1.0 * M
    loss_cls = jnp.mean((base[:, 0, 0] + res[:, 0]) / denom)
    loss_bbox = jnp.mean(res[:, 1] / (M * 4)) * BBOX_W
    loss_ang = jnp.mean(res[:, 2] / M) * ANG_W
    return (loss_cls + loss_bbox + loss_ang, loss_cls, loss_bbox, loss_ang)


# speedup vs baseline: 63.8988x; 1.0240x over previous
import functools

import jax
import jax.numpy as jnp
from jax import lax
from jax.experimental import pallas as pl
from jax.experimental.pallas import tpu as pltpu
from jax.experimental.pallas import tpu_sc as plsc

NCLS = 15
CLS_W = 2.0
BBOX_W = 5.0
ANG_W = 2.0
NOOBJ_W = 0.1
B, Q, M = 4, 1000, 200
QP, MP = 1024, 256
NPP = NCLS + 1 + 5
LANES = 16
BIGI = 2 ** 30

_COS_C = [
    1.0, -0.5, 1.0 / 24, -1.0 / 720, 1.0 / 40320, -1.0 / 3628800,
    1.0 / 479001600, -1.0 / 87178291200,
]


def _cos_scalar(x):
    t = x * x
    r = jnp.float32(_COS_C[7])
    for k in range(6, -1, -1):
        r = r * t + jnp.float32(_COS_C[k])
    return r


def _tc_body(size_ref, lg_ref, pb_ref, tb_ref, lab_ref,
             cost_ref, colmin_ref, colargq_ref, pp_ref, tg_ref, base_ref):
    zq = jnp.zeros((16, QP - Q), jnp.float32)
    lt = jnp.concatenate([jnp.swapaxes(lg_ref[0], 0, 1), zq], axis=1)
    pbt = jnp.concatenate([jnp.swapaxes(pb_ref[0], 0, 1), zq[:5]], axis=1)
    mx = jnp.max(lt, axis=0, keepdims=True)
    ex = jnp.exp(lt - mx)
    s = jnp.sum(ex, axis=0, keepdims=True)
    logp = lt - mx - jnp.log(s)
    pp_ref[0, :NCLS + 1, :] = logp
    pp_ref[0, NCLS + 1:, :] = pbt
    prob = ex / s

    hh = size_ref[0, 0, 0].astype(jnp.float32)
    ww = size_ref[0, 0, 1].astype(jnp.float32)
    tbr = tb_ref[0]
    labr = lab_ref[0]
    tg_ref[0, :, :5] = tbr
    tg_ref[0, :, 5:6] = labr.astype(jnp.float32)
    tg_ref[0, :, 6:] = jnp.zeros((M, 2), jnp.float32)
    tb = jnp.concatenate([tbr, jnp.zeros((MP - M, 5), jnp.float32)], axis=0)
    lab0 = jnp.concatenate(
        [labr - 1, jnp.full((MP - M, 1), NCLS, jnp.int32)], axis=0)

    sel = [prob[c:c + 1, :] for c in range(16)]
    for bit in (1, 2, 4, 8):
        cond = (lab0 & bit) != 0
        sel = [jnp.where(cond, sel[i + 1], sel[i])
               for i in range(0, len(sel), 2)]
    cls_cost = sel[0] * (-CLS_W)

    di = lax.broadcasted_iota(jnp.int32, (1, 5), 1)
    scale5 = jnp.where((di == 0) | (di == 2), ww,
                       jnp.where(di == 4, 1.0, hh))
    tbn = tb / scale5
    l1 = jnp.abs(pbt[0:1, :] - tbn[:, 0:1])
    for d in range(1, 4):
        l1 = l1 + jnp.abs(pbt[d:d + 1, :] - tbn[:, d:d + 1])
    pth = pbt[4:5, :]
    tth = tb[:, 4:5]
    ang = 1.0 - (jnp.cos(pth) * jnp.cos(tth) + jnp.sin(pth) * jnp.sin(tth))
    cost = cls_cost + l1 * BBOX_W + ang * ANG_W

    qi = lax.broadcasted_iota(jnp.int32, (MP, QP), 1)
    mi = lax.broadcasted_iota(jnp.int32, (MP, QP), 0)
    cost = jnp.where((qi >= Q) | (mi >= M), jnp.inf, cost)
    cost_ref[0] = cost[:M]

    cmin = jnp.min(cost, axis=1, keepdims=True)
    colmin_ref[0] = cmin
    ismin = cost == cmin
    colargq_ref[0] = jnp.min(jnp.where(ismin, qi, QP), axis=1, keepdims=True)

    row15 = logp[NCLS:NCLS + 1, :]
    qrow = lax.broadcasted_iota(jnp.int32, (1, QP), 1)
    base_ref[0, 0, 0] = NOOBJ_W * jnp.sum(jnp.where(qrow < Q, -row15, 0.0))


def _tc_stage(size, lg, pb, tbr, labr):
    f32 = jnp.float32
    out_shapes = (
        jax.ShapeDtypeStruct((B, M, QP), f32),
        jax.ShapeDtypeStruct((B, MP, 1), f32),
        jax.ShapeDtypeStruct((B, MP, 1), jnp.int32),
        jax.ShapeDtypeStruct((B, NPP, QP), f32),
        jax.ShapeDtypeStruct((B, M, 8), f32),
        jax.ShapeDtypeStruct((B, 1, 1), f32),
    )
    grid = (B,)
    return pl.pallas_call(
        _tc_body,
        grid=grid,
        in_specs=[
            pl.BlockSpec((1, 1, 2), lambda b: (b, 0, 0), memory_space=pltpu.SMEM),
            pl.BlockSpec((1, Q, 16), lambda b: (b, 0, 0)),
            pl.BlockSpec((1, Q, 5), lambda b: (b, 0, 0)),
            pl.BlockSpec((1, M, 5), lambda b: (b, 0, 0)),
            pl.BlockSpec((1, M, 1), lambda b: (b, 0, 0)),
        ],
        out_specs=[
            pl.BlockSpec((1, M, QP), lambda b: (b, 0, 0)),
            pl.BlockSpec((1, MP, 1), lambda b: (b, 0, 0)),
            pl.BlockSpec((1, MP, 1), lambda b: (b, 0, 0)),
            pl.BlockSpec((1, NPP, QP), lambda b: (b, 0, 0)),
            pl.BlockSpec((1, M, 8), lambda b: (b, 0, 0)),
            pl.BlockSpec((1, 1, 1), lambda b: (b, 0, 0), memory_space=pltpu.SMEM),
        ],
        out_shape=out_shapes,
    )(size, lg, pb, tbr, labr)


def _sc_greedy(cost_hbm, colmin_hbm, colargq_hbm, pp_hbm, tg_hbm, size_hbm,
               out_hbm,
               colmin_v, colargq_v, pp_v, tg_v, size_v,
               qmask_v, rowbuf_v, outbuf_v, summary_v, mqm_v, cost_sh, dsem):
    info = plsc.get_sparse_core_info()
    ns = info.num_subcores
    wid = lax.axis_index("c") * ns + lax.axis_index("s")

    iota16 = lax.broadcasted_iota(jnp.int32, (LANES,), 0)
    lane0 = iota16 == 0

    def _gat(ref, *idx):
        idxs = [jnp.broadcast_to(i, (LANES,)).astype(jnp.int32) for i in idx]
        return plsc.load_gather(ref, idxs)[0]

    def _gatv(ref, idx16):
        return plsc.load_gather(ref, [idx16])

    def _put(ref, i, val):
        ii = jnp.broadcast_to(i, (LANES,)).astype(jnp.int32)
        plsc.store_scatter(ref, [ii], jnp.broadcast_to(val, (LANES,)),
                           mask=lane0)

    @pl.when(wid < B)
    def _work():
        b = wid
        with jax.named_scope("sc_stage_in"):
            stage = pltpu.async_copy(cost_hbm.at[b], cost_sh.at[b], dsem)
            pltpu.sync_copy(colmin_hbm.at[b], colmin_v)
            pltpu.sync_copy(colargq_hbm.at[b], colargq_v)
            pltpu.sync_copy(pp_hbm.at[b], pp_v)
            pltpu.sync_copy(tg_hbm.at[b], tg_v)
            pltpu.sync_copy(size_hbm.at[b], size_v)

        zeros16 = jnp.zeros((LANES,), jnp.float32)
        for k in range(QP // LANES):
            qmask_v[pl.ds(k * LANES, LANES)] = zeros16
        for k in range(MP // LANES):
            _put(summary_v, k, jnp.min(colmin_v[pl.ds(k * LANES, LANES)]))
        mqm_v[pl.ds(M - 8, LANES)] = jnp.zeros((LANES,), jnp.int32)

        sizes = size_v[pl.ds(0, LANES)]
        rcp = 1.0 / sizes.astype(jnp.float32)
        rw = rcp[1]
        rh = rcp[0]
        rs = (rw, rh, rw, rh)
        inf = jnp.float32(jnp.inf)

        def upd_summary(m):
            k = lax.shift_right_logical(m, 4)
            _put(summary_v, k, jnp.min(colmin_v[pl.ds(k * LANES, LANES)]))

        def recompute_col(m2):
            pltpu.sync_copy(cost_sh.at[b, m2], rowbuf_v)
            bv = rowbuf_v[pl.ds(0, LANES)] + qmask_v[pl.ds(0, LANES)]
            bq = iota16
            for k in range(1, QP // LANES):
                v = rowbuf_v[pl.ds(k * LANES, LANES)] + qmask_v[pl.ds(k * LANES, LANES)]
                qv = iota16 + (k * LANES)
                lt2 = (v < bv) | ((v == bv) & (qv < bq))
                bv = jnp.where(lt2, v, bv)
                bq = jnp.where(lt2, qv, bq)
            mv = jnp.min(bv)
            _put(colmin_v, m2, mv)
            _put(colargq_v, m2, jnp.min(jnp.where(bv == mv, bq, BIGI)))
            upd_summary(m2)

        def full_scan():
            bv = colmin_v[pl.ds(0, LANES)]
            bq = colargq_v[pl.ds(0, LANES)]
            bm = iota16
            for k in range(1, MP // LANES):
                v = colmin_v[pl.ds(k * LANES, LANES)]
                qv = colargq_v[pl.ds(k * LANES, LANES)]
                mv_ = iota16 + (k * LANES)
                lt2 = (v < bv) | ((v == bv) & ((qv < bq) | ((qv == bq) & (mv_ < bm))))
                bv = jnp.where(lt2, v, bv)
                bq = jnp.where(lt2, qv, bq)
                bm = jnp.where(lt2, mv_, bm)
            gv = jnp.min(bv)
            c1 = bv == gv
            gq = jnp.min(jnp.where(c1, bq, BIGI))
            gm = jnp.min(jnp.where(c1 & (bq == gq), bm, BIGI))
            return gv, gq, gm

        def scan_min():
            sv = summary_v[pl.ds(0, LANES)]
            sk, skidx = plsc.sort_key_val(sv, iota16)
            k0 = skidx[0]
            off = k0 * LANES
            v = colmin_v[pl.ds(off, LANES)]
            qv = colargq_v[pl.ds(off, LANES)]
            ck, cp = plsc.sort_key_val(v, qv * LANES + iota16)
            p0 = cp[0]
            tie = (sk[1] == sk[0]) | (ck[1] == ck[0])

            def slow():
                gv, gq, gm = full_scan()
                return gv, gq, gm, jnp.int32(0), jnp.float32(0.0)

            return lax.cond(
                tie, slow,
                lambda: (ck[0], lax.shift_right_logical(p0, 4),
                         off + (p0 & (LANES - 1)), jnp.int32(1), ck[1]))

        def body(it, carry):
            def wcond(st):
                return _gat(qmask_v, st[1]) != 0.0

            def wbody(st):
                recompute_col(st[2])
                return scan_min()

            gv, gq, gm, fast, alt = lax.while_loop(wcond, wbody, scan_min())

            _put(mqm_v, it, gq * MP + gm)

            _put(colmin_v, gm, inf)
            _put(qmask_v, gq, inf)
            k = lax.shift_right_logical(gm, 4)
            newmin = lax.cond(
                fast == 1, lambda: alt,
                lambda: jnp.min(colmin_v[pl.ds(k * LANES, LANES)]))
            _put(summary_v, k, newmin)
            return carry

        with jax.named_scope("sc_match"):
            stage.wait()
            lax.fori_loop(0, M, body, 0)

        z16 = jnp.zeros((LANES,), jnp.float32)
        ccorr_v = z16
        bsum_v = z16
        asum_v = z16
        for g in range(MP // LANES):
            base_i = g * LANES
            if base_i >= M:
                break
            pk = mqm_v[pl.ds(base_i, LANES)]
            q16 = lax.shift_right_logical(pk, 8)
            m16 = pk & (MP - 1)
            labm16 = _gatv(tg_v, m16 * 8 + 5).astype(jnp.int32) - 1
            lp = _gatv(pp_v, labm16 * QP + q16)
            lp15 = _gatv(pp_v, NCLS * QP + q16)
            t = [_gatv(tg_v, m16 * 8 + d) for d in range(5)]
            p = [_gatv(pp_v, (NCLS + 1 + d) * QP + q16) for d in range(5)]
            cc = -lp + NOOBJ_W * lp15
            l1 = jnp.abs(p[0] - t[0] * rs[0])
            for d in range(1, 4):
                l1 = l1 + jnp.abs(p[d] - t[d] * rs[d])
            dth = p[4] - t[4]
            av = 1.0 - _cos_scalar(dth)
            if base_i + LANES > M:
                valid = iota16 < (M - base_i)
                cc = jnp.where(valid, cc, 0.0)
                l1 = jnp.where(valid, l1, 0.0)
                av = jnp.where(valid, av, 0.0)
            ccorr_v = ccorr_v + cc
            bsum_v = bsum_v + l1
            asum_v = asum_v + av
        ccorr = jnp.sum(ccorr_v)
        bsum = jnp.sum(bsum_v)
        asum = jnp.sum(asum_v)
        out16 = jnp.where(iota16 == 0, ccorr,
                          jnp.where(iota16 == 1, bsum,
                                    jnp.where(iota16 == 2, asum, 0.0)))
        outbuf_v[pl.ds(0, LANES)] = out16
        pltpu.sync_copy(outbuf_v, out_hbm.at[b])


def _sc_stage(cost, colmin, colargq, pp, tg, size):
    mesh = plsc.VectorSubcoreMesh(core_axis_name="c", subcore_axis_name="s")
    f32 = jnp.float32
    fn = functools.partial(
        pl.kernel,
        mesh=mesh,
        compiler_params=pltpu.CompilerParams(needs_layout_passes=False),
        out_type=jax.ShapeDtypeStruct((B, 16), f32),
        scratch_types=[
            pltpu.VMEM((MP,), f32),
            pltpu.VMEM((MP,), jnp.int32),
            pltpu.VMEM((NPP * QP,), f32),
            pltpu.VMEM((M * 8,), f32),
            pltpu.VMEM((LANES,), jnp.int32),
            pltpu.VMEM((QP,), f32),
            pltpu.VMEM((QP,), f32),
            pltpu.VMEM((LANES,), f32),
            pltpu.VMEM((LANES,), f32),
            pltpu.VMEM((MP,), jnp.int32),
            pltpu.VMEM_SHARED((B, M, QP), f32),
            pltpu.SemaphoreType.DMA,
        ],
    )(_sc_greedy)
    return fn(cost, colmin, colargq, pp, tg, size)


@jax.jit
def kernel(pred_logits, pred_boxes, tgt_boxes, tgt_labels, tgt_size):
    f32 = jnp.float32
    size32 = tgt_size.astype(jnp.int32)
    sizep = jnp.pad(size32, ((0, 0), (0, 14)))

    cost, colmin, colargq, pp, tg, base = _tc_stage(
        size32[:, None, :], pred_logits.astype(f32), pred_boxes.astype(f32),
        tgt_boxes.astype(f32), tgt_labels.astype(jnp.int32)[..., None])

    res = _sc_stage(cost, colmin.reshape(B, MP), colargq.reshape(B, MP),
                    pp.reshape(B, NPP * QP), tg.reshape(B, M * 8), sizep)

    denom = NOOBJ_W * (Q - M) + 1.0 * M
    loss_cls = jnp.mean((base[:, 0, 0] + res[:, 0]) / denom)
    loss_bbox = jnp.mean(res[:, 1] / (M * 4)) * BBOX_W
    loss_ang = jnp.mean(res[:, 2] / M) * ANG_W
    return (loss_cls + loss_bbox + loss_ang, loss_cls, loss_bbox, loss_ang)


# speedup vs baseline: 67.2792x; 1.0529x over previous
import functools

import jax
import jax.numpy as jnp
from jax import lax
from jax.experimental import pallas as pl
from jax.experimental.pallas import tpu as pltpu
from jax.experimental.pallas import tpu_sc as plsc

NCLS = 15
CLS_W = 2.0
BBOX_W = 5.0
ANG_W = 2.0
NOOBJ_W = 0.1
B, Q, M = 4, 1000, 200
QP, MP = 1024, 256
NPP = NCLS + 1 + 5
LANES = 16
BIGI = 2 ** 30

_COS_C = [
    1.0, -0.5, 1.0 / 24, -1.0 / 720, 1.0 / 40320, -1.0 / 3628800,
    1.0 / 479001600, -1.0 / 87178291200,
]


def _cos_scalar(x):
    t = x * x
    r = jnp.float32(_COS_C[7])
    for k in range(6, -1, -1):
        r = r * t + jnp.float32(_COS_C[k])
    return r


def _tc_body(size_ref, lg_ref, pb_ref, tb_ref, lab_ref,
             cost_ref, colmin_ref, colargq_ref, pp_ref, tg_ref, base_ref):
    zq = jnp.zeros((16, QP - Q), jnp.float32)
    lt = jnp.concatenate([jnp.swapaxes(lg_ref[0], 0, 1), zq], axis=1)
    pbt = jnp.concatenate([jnp.swapaxes(pb_ref[0], 0, 1), zq[:5]], axis=1)
    mx = jnp.max(lt, axis=0, keepdims=True)
    ex = jnp.exp(lt - mx)
    s = jnp.sum(ex, axis=0, keepdims=True)
    logp = lt - mx - jnp.log(s)
    pp_ref[0, :NCLS + 1, :] = logp
    pp_ref[0, NCLS + 1:, :] = pbt
    prob = ex / s

    hh = size_ref[0, 0, 0].astype(jnp.float32)
    ww = size_ref[0, 0, 1].astype(jnp.float32)
    tbr = tb_ref[0]
    labr = lab_ref[0]
    tg_ref[0, :, :5] = tbr
    tg_ref[0, :, 5:6] = labr.astype(jnp.float32)
    tg_ref[0, :, 6:] = jnp.zeros((M, 2), jnp.float32)
    tb = jnp.concatenate([tbr, jnp.zeros((MP - M, 5), jnp.float32)], axis=0)
    lab0 = jnp.concatenate(
        [labr - 1, jnp.full((MP - M, 1), NCLS, jnp.int32)], axis=0)

    sel = [prob[c:c + 1, :] for c in range(16)]
    for bit in (1, 2, 4, 8):
        cond = (lab0 & bit) != 0
        sel = [jnp.where(cond, sel[i + 1], sel[i])
               for i in range(0, len(sel), 2)]
    cls_cost = sel[0] * (-CLS_W)

    di = lax.broadcasted_iota(jnp.int32, (1, 5), 1)
    scale5 = jnp.where((di == 0) | (di == 2), ww,
                       jnp.where(di == 4, 1.0, hh))
    tbn = tb / scale5
    l1 = jnp.abs(pbt[0:1, :] - tbn[:, 0:1])
    for d in range(1, 4):
        l1 = l1 + jnp.abs(pbt[d:d + 1, :] - tbn[:, d:d + 1])
    pth = pbt[4:5, :]
    tth = tb[:, 4:5]
    ang = 1.0 - (jnp.cos(pth) * jnp.cos(tth) + jnp.sin(pth) * jnp.sin(tth))
    cost = cls_cost + l1 * BBOX_W + ang * ANG_W

    qi = lax.broadcasted_iota(jnp.int32, (MP, QP), 1)
    mi = lax.broadcasted_iota(jnp.int32, (MP, QP), 0)
    cost = jnp.where((qi >= Q) | (mi >= M), jnp.inf, cost)
    cost_ref[0] = cost[:M]

    cmin = jnp.min(cost, axis=1, keepdims=True)
    colmin_ref[0] = cmin
    ismin = cost == cmin
    colargq_ref[0] = jnp.min(jnp.where(ismin, qi, QP), axis=1, keepdims=True)

    row15 = logp[NCLS:NCLS + 1, :]
    qrow = lax.broadcasted_iota(jnp.int32, (1, QP), 1)
    base_ref[0, 0, 0] = NOOBJ_W * jnp.sum(jnp.where(qrow < Q, -row15, 0.0))


def _tc_stage(size, lg, pb, tbr, labr):
    f32 = jnp.float32
    out_shapes = (
        jax.ShapeDtypeStruct((B, M, QP), f32),
        jax.ShapeDtypeStruct((B, MP, 1), f32),
        jax.ShapeDtypeStruct((B, MP, 1), jnp.int32),
        jax.ShapeDtypeStruct((B, NPP, QP), f32),
        jax.ShapeDtypeStruct((B, M, 8), f32),
        jax.ShapeDtypeStruct((B, 1, 1), f32),
    )
    grid = (B,)
    return pl.pallas_call(
        _tc_body,
        grid=grid,
        in_specs=[
            pl.BlockSpec((1, 1, 2), lambda b: (b, 0, 0), memory_space=pltpu.SMEM),
            pl.BlockSpec((1, Q, 16), lambda b: (b, 0, 0)),
            pl.BlockSpec((1, Q, 5), lambda b: (b, 0, 0)),
            pl.BlockSpec((1, M, 5), lambda b: (b, 0, 0)),
            pl.BlockSpec((1, M, 1), lambda b: (b, 0, 0)),
        ],
        out_specs=[
            pl.BlockSpec((1, M, QP), lambda b: (b, 0, 0)),
            pl.BlockSpec((1, MP, 1), lambda b: (b, 0, 0)),
            pl.BlockSpec((1, MP, 1), lambda b: (b, 0, 0)),
            pl.BlockSpec((1, NPP, QP), lambda b: (b, 0, 0)),
            pl.BlockSpec((1, M, 8), lambda b: (b, 0, 0)),
            pl.BlockSpec((1, 1, 1), lambda b: (b, 0, 0), memory_space=pltpu.SMEM),
        ],
        out_shape=out_shapes,
    )(size, lg, pb, tbr, labr)


def _sc_greedy(cost_hbm, colmin_hbm, colargq_hbm, pp_hbm, tg_hbm, size_hbm,
               out_hbm,
               colmin_v, colargq_v, pp_v, tg_v, size_v,
               qmask_v, rowbuf_v, outbuf_v, summary_v, mqm_v, cost_sh, dsem):
    info = plsc.get_sparse_core_info()
    ns = info.num_subcores
    wid = lax.axis_index("c") * ns + lax.axis_index("s")

    iota16 = lax.broadcasted_iota(jnp.int32, (LANES,), 0)
    lane0 = iota16 == 0

    def _gat(ref, *idx):
        idxs = [jnp.broadcast_to(i, (LANES,)).astype(jnp.int32) for i in idx]
        return plsc.load_gather(ref, idxs)[0]

    def _gatv(ref, idx16):
        return plsc.load_gather(ref, [idx16])

    def _put(ref, i, val):
        ii = jnp.broadcast_to(i, (LANES,)).astype(jnp.int32)
        plsc.store_scatter(ref, [ii], jnp.broadcast_to(val, (LANES,)),
                           mask=lane0)

    @pl.when(wid < B)
    def _work():
        b = wid
        with jax.named_scope("sc_stage_in"):
            stage = pltpu.async_copy(cost_hbm.at[b], cost_sh.at[b], dsem)
            pltpu.sync_copy(colmin_hbm.at[b], colmin_v)
            pltpu.sync_copy(colargq_hbm.at[b], colargq_v)
            pltpu.sync_copy(pp_hbm.at[b], pp_v)
            pltpu.sync_copy(tg_hbm.at[b], tg_v)
            pltpu.sync_copy(size_hbm.at[b], size_v)

        zeros16 = jnp.zeros((LANES,), jnp.float32)
        for k in range(QP // LANES):
            qmask_v[pl.ds(k * LANES, LANES)] = zeros16
        for k in range(MP // LANES):
            _put(summary_v, k, jnp.min(colmin_v[pl.ds(k * LANES, LANES)]))
        mqm_v[pl.ds(M - 8, LANES)] = jnp.zeros((LANES,), jnp.int32)

        sizes = size_v[pl.ds(0, LANES)]
        rcp = 1.0 / sizes.astype(jnp.float32)
        rw = rcp[1]
        rh = rcp[0]
        rs = (rw, rh, rw, rh)
        inf = jnp.float32(jnp.inf)

        def upd_summary(m):
            k = lax.shift_right_logical(m, 4)
            _put(summary_v, k, jnp.min(colmin_v[pl.ds(k * LANES, LANES)]))

        def recompute_col(m2):
            pltpu.sync_copy(cost_sh.at[b, m2], rowbuf_v)
            bv = rowbuf_v[pl.ds(0, LANES)] + qmask_v[pl.ds(0, LANES)]
            bq = iota16
            for k in range(1, QP // LANES):
                v = rowbuf_v[pl.ds(k * LANES, LANES)] + qmask_v[pl.ds(k * LANES, LANES)]
                qv = iota16 + (k * LANES)
                lt2 = (v < bv) | ((v == bv) & (qv < bq))
                bv = jnp.where(lt2, v, bv)
                bq = jnp.where(lt2, qv, bq)
            mv = jnp.min(bv)
            _put(colmin_v, m2, mv)
            _put(colargq_v, m2, jnp.min(jnp.where(bv == mv, bq, BIGI)))
            upd_summary(m2)

        def full_scan():
            bv = colmin_v[pl.ds(0, LANES)]
            bq = colargq_v[pl.ds(0, LANES)]
            bm = iota16
            for k in range(1, MP // LANES):
                v = colmin_v[pl.ds(k * LANES, LANES)]
                qv = colargq_v[pl.ds(k * LANES, LANES)]
                mv_ = iota16 + (k * LANES)
                lt2 = (v < bv) | ((v == bv) & ((qv < bq) | ((qv == bq) & (mv_ < bm))))
                bv = jnp.where(lt2, v, bv)
                bq = jnp.where(lt2, qv, bq)
                bm = jnp.where(lt2, mv_, bm)
            gv = jnp.min(bv)
            c1 = bv == gv
            gq = jnp.min(jnp.where(c1, bq, BIGI))
            gm = jnp.min(jnp.where(c1 & (bq == gq), bm, BIGI))
            return gv, gq, gm

        def scan_min():
            sv = summary_v[pl.ds(0, LANES)]
            sk, skidx = plsc.sort_key_val(sv, iota16)
            k0 = skidx[0]
            off = k0 * LANES
            v = colmin_v[pl.ds(off, LANES)]
            qv = colargq_v[pl.ds(off, LANES)]
            ck, cp = plsc.sort_key_val(v, qv * LANES + iota16)
            p0 = cp[0]
            tie = (sk[1] == sk[0]) | (ck[1] == ck[0])

            def slow():
                gv, gq, gm = full_scan()
                ks = lax.shift_right_logical(gm, 4)
                cv = colmin_v[pl.ds(ks * LANES, LANES)]
                cv = jnp.where(iota16 == (gm & (LANES - 1)), jnp.inf, cv)
                return gv, gq, gm, jnp.min(cv)

            return lax.cond(
                tie, slow,
                lambda: (ck[0], lax.shift_right_logical(p0, 4),
                         off + (p0 & (LANES - 1)), ck[1]))

        def body(it, carry):
            def wcond(st):
                return _gat(qmask_v, st[1]) != 0.0

            def wbody(st):
                recompute_col(st[2])
                return scan_min()

            gv, gq, gm, alt = lax.while_loop(wcond, wbody, scan_min())

            _put(mqm_v, it, gq * MP + gm)

            _put(colmin_v, gm, inf)
            _put(qmask_v, gq, inf)
            _put(summary_v, lax.shift_right_logical(gm, 4), alt)
            return carry

        with jax.named_scope("sc_match"):
            stage.wait()
            lax.fori_loop(0, M, body, 0, unroll=2)

        z16 = jnp.zeros((LANES,), jnp.float32)
        ccorr_v = z16
        bsum_v = z16
        asum_v = z16
        for g in range(MP // LANES):
            base_i = g * LANES
            if base_i >= M:
                break
            pk = mqm_v[pl.ds(base_i, LANES)]
            q16 = lax.shift_right_logical(pk, 8)
            m16 = pk & (MP - 1)
            c5 = jnp.broadcast_to(jnp.int32(5), (LANES,))
            labm16 = plsc.load_gather(tg_v, [m16, c5]).astype(jnp.int32) - 1
            lp = plsc.load_gather(pp_v, [labm16, q16])
            lp15 = plsc.load_gather(pp_v, [c5 + (NCLS - 5), q16])
            t = [plsc.load_gather(tg_v, [m16, jnp.broadcast_to(jnp.int32(d), (LANES,))])
                 for d in range(5)]
            p = [plsc.load_gather(pp_v, [jnp.broadcast_to(jnp.int32(NCLS + 1 + d), (LANES,)), q16])
                 for d in range(5)]
            cc = -lp + NOOBJ_W * lp15
            l1 = jnp.abs(p[0] - t[0] * rs[0])
            for d in range(1, 4):
                l1 = l1 + jnp.abs(p[d] - t[d] * rs[d])
            dth = p[4] - t[4]
            av = 1.0 - _cos_scalar(dth)
            if base_i + LANES > M:
                valid = iota16 < (M - base_i)
                cc = jnp.where(valid, cc, 0.0)
                l1 = jnp.where(valid, l1, 0.0)
                av = jnp.where(valid, av, 0.0)
            ccorr_v = ccorr_v + cc
            bsum_v = bsum_v + l1
            asum_v = asum_v + av
        ccorr = jnp.sum(ccorr_v)
        bsum = jnp.sum(bsum_v)
        asum = jnp.sum(asum_v)
        out16 = jnp.where(iota16 == 0, ccorr,
                          jnp.where(iota16 == 1, bsum,
                                    jnp.where(iota16 == 2, asum, 0.0)))
        outbuf_v[pl.ds(0, LANES)] = out16
        pltpu.sync_copy(outbuf_v, out_hbm.at[b])


def _sc_stage(cost, colmin, colargq, pp, tg, size):
    mesh = plsc.VectorSubcoreMesh(core_axis_name="c", subcore_axis_name="s")
    f32 = jnp.float32
    fn = functools.partial(
        pl.kernel,
        mesh=mesh,
        compiler_params=pltpu.CompilerParams(needs_layout_passes=False),
        out_type=jax.ShapeDtypeStruct((B, 16), f32),
        scratch_types=[
            pltpu.VMEM((MP,), f32),
            pltpu.VMEM((MP,), jnp.int32),
            pltpu.VMEM((NPP, QP), f32),
            pltpu.VMEM((M, 8), f32),
            pltpu.VMEM((LANES,), jnp.int32),
            pltpu.VMEM((QP,), f32),
            pltpu.VMEM((QP,), f32),
            pltpu.VMEM((LANES,), f32),
            pltpu.VMEM((LANES,), f32),
            pltpu.VMEM((MP,), jnp.int32),
            pltpu.VMEM_SHARED((B, M, QP), f32),
            pltpu.SemaphoreType.DMA,
        ],
    )(_sc_greedy)
    return fn(cost, colmin, colargq, pp, tg, size)


@jax.jit
def kernel(pred_logits, pred_boxes, tgt_boxes, tgt_labels, tgt_size):
    f32 = jnp.float32
    size32 = tgt_size.astype(jnp.int32)
    sizep = jnp.pad(size32, ((0, 0), (0, 14)))

    cost, colmin, colargq, pp, tg, base = _tc_stage(
        size32[:, None, :], pred_logits.astype(f32), pred_boxes.astype(f32),
        tgt_boxes.astype(f32), tgt_labels.astype(jnp.int32)[..., None])

    res = _sc_stage(cost, colmin.reshape(B, MP), colargq.reshape(B, MP),
                    pp, tg, sizep)

    denom = NOOBJ_W * (Q - M) + 1.0 * M
    loss_cls = jnp.mean((base[:, 0, 0] + res[:, 0]) / denom)
    loss_bbox = jnp.mean(res[:, 1] / (M * 4)) * BBOX_W
    loss_ang = jnp.mean(res[:, 2] / M) * ANG_W
    return (loss_cls + loss_bbox + loss_ang, loss_cls, loss_bbox, loss_ang)


# speedup vs baseline: 72.9428x; 1.0842x over previous
import functools

import jax
import jax.numpy as jnp
from jax import lax
from jax.experimental import pallas as pl
from jax.experimental.pallas import tpu as pltpu
from jax.experimental.pallas import tpu_sc as plsc

NCLS = 15
CLS_W = 2.0
BBOX_W = 5.0
ANG_W = 2.0
NOOBJ_W = 0.1
B, Q, M = 4, 1000, 200
QP, MP = 1024, 256
NPP = NCLS + 1 + 5
LANES = 16
BIGI = 2 ** 30

_COS_C = [
    1.0, -0.5, 1.0 / 24, -1.0 / 720, 1.0 / 40320, -1.0 / 3628800,
    1.0 / 479001600, -1.0 / 87178291200,
]


def _cos_scalar(x):
    t = x * x
    r = jnp.float32(_COS_C[7])
    for k in range(6, -1, -1):
        r = r * t + jnp.float32(_COS_C[k])
    return r


def _tc_body(size_ref, lg_ref, pb_ref, tb_ref, lab_ref,
             cost_ref, colmin_ref, colargq_ref, pp_ref, tg_ref, base_ref):
    zq = jnp.zeros((16, QP - Q), jnp.float32)
    lt = jnp.concatenate([lg_ref[0], zq], axis=1)
    pbt = jnp.concatenate([pb_ref[0], zq[:5]], axis=1)
    mx = jnp.max(lt, axis=0, keepdims=True)
    ex = jnp.exp(lt - mx)
    s = jnp.sum(ex, axis=0, keepdims=True)
    logp = lt - mx - jnp.log(s)
    pp_ref[0, :NCLS + 1, :] = logp
    pp_ref[0, NCLS + 1:, :] = pbt
    prob = ex / s

    hh = size_ref[0, 0, 0].astype(jnp.float32)
    ww = size_ref[0, 0, 1].astype(jnp.float32)
    tbr = tb_ref[0]
    labr = lab_ref[0]
    tg_ref[0, :, :5] = tbr
    tg_ref[0, :, 5:6] = labr.astype(jnp.float32)
    tg_ref[0, :, 6:] = jnp.zeros((M, 2), jnp.float32)
    tb = jnp.concatenate([tbr, jnp.zeros((MP - M, 5), jnp.float32)], axis=0)
    lab0 = jnp.concatenate(
        [labr - 1, jnp.full((MP - M, 1), NCLS, jnp.int32)], axis=0)

    sel = [prob[c:c + 1, :] for c in range(16)]
    for bit in (1, 2, 4, 8):
        cond = (lab0 & bit) != 0
        sel = [jnp.where(cond, sel[i + 1], sel[i])
               for i in range(0, len(sel), 2)]
    cls_cost = sel[0] * (-CLS_W)

    di = lax.broadcasted_iota(jnp.int32, (1, 5), 1)
    scale5 = jnp.where((di == 0) | (di == 2), ww,
                       jnp.where(di == 4, 1.0, hh))
    tbn = tb / scale5
    l1 = jnp.abs(pbt[0:1, :] - tbn[:, 0:1])
    for d in range(1, 4):
        l1 = l1 + jnp.abs(pbt[d:d + 1, :] - tbn[:, d:d + 1])
    pth = pbt[4:5, :]
    tth = tb[:, 4:5]
    ang = 1.0 - (jnp.cos(pth) * jnp.cos(tth) + jnp.sin(pth) * jnp.sin(tth))
    cost = cls_cost + l1 * BBOX_W + ang * ANG_W

    qi = lax.broadcasted_iota(jnp.int32, (MP, QP), 1)
    mi = lax.broadcasted_iota(jnp.int32, (MP, QP), 0)
    cost = jnp.where((qi >= Q) | (mi >= M), jnp.inf, cost)
    cost_ref[0] = cost[:M]

    cmin = jnp.min(cost, axis=1, keepdims=True)
    colmin_ref[0] = jnp.swapaxes(cmin, 0, 1)
    ismin = cost == cmin
    argq = jnp.min(jnp.where(ismin, qi, QP), axis=1, keepdims=True)
    colargq_ref[0] = jnp.swapaxes(argq, 0, 1)

    row15 = logp[NCLS:NCLS + 1, :]
    qrow = lax.broadcasted_iota(jnp.int32, (1, QP), 1)
    base_ref[0, 0, 0] = NOOBJ_W * jnp.sum(jnp.where(qrow < Q, -row15, 0.0))


def _tc_stage(size, lg, pb, tbr, labr):
    f32 = jnp.float32
    out_shapes = (
        jax.ShapeDtypeStruct((B, M, QP), f32),
        jax.ShapeDtypeStruct((B, 1, MP), f32),
        jax.ShapeDtypeStruct((B, 1, MP), jnp.int32),
        jax.ShapeDtypeStruct((B, NPP, QP), f32),
        jax.ShapeDtypeStruct((B, M, 8), f32),
        jax.ShapeDtypeStruct((B, 1, 1), f32),
    )
    grid = (B,)
    return pl.pallas_call(
        _tc_body,
        grid=grid,
        in_specs=[
            pl.BlockSpec((1, 1, 2), lambda b: (b, 0, 0), memory_space=pltpu.SMEM),
            pl.BlockSpec((1, 16, Q), lambda b: (b, 0, 0)),
            pl.BlockSpec((1, 5, Q), lambda b: (b, 0, 0)),
            pl.BlockSpec((1, M, 5), lambda b: (b, 0, 0)),
            pl.BlockSpec((1, M, 1), lambda b: (b, 0, 0)),
        ],
        out_specs=[
            pl.BlockSpec((1, M, QP), lambda b: (b, 0, 0)),
            pl.BlockSpec((1, 1, MP), lambda b: (b, 0, 0)),
            pl.BlockSpec((1, 1, MP), lambda b: (b, 0, 0)),
            pl.BlockSpec((1, NPP, QP), lambda b: (b, 0, 0)),
            pl.BlockSpec((1, M, 8), lambda b: (b, 0, 0)),
            pl.BlockSpec((1, 1, 1), lambda b: (b, 0, 0), memory_space=pltpu.SMEM),
        ],
        out_shape=out_shapes,
    )(size, lg, pb, tbr, labr)


def _sc_greedy(cost_hbm, colmin_hbm, colargq_hbm, pp_hbm, tg_hbm, size_hbm,
               out_hbm,
               colmin_v, colargq_v, pp_v, tg_v, size_v,
               qmask_v, rowbuf_v, outbuf_v, summary_v, mqm_v, cost_sh, dsem,
               dsem2):
    info = plsc.get_sparse_core_info()
    ns = info.num_subcores
    wid = lax.axis_index("c") * ns + lax.axis_index("s")

    iota16 = lax.broadcasted_iota(jnp.int32, (LANES,), 0)
    lane0 = iota16 == 0

    def _gat(ref, *idx):
        idxs = [jnp.broadcast_to(i, (LANES,)).astype(jnp.int32) for i in idx]
        return plsc.load_gather(ref, idxs)[0]

    def _gatv(ref, idx16):
        return plsc.load_gather(ref, [idx16])

    def _put(ref, i, val):
        ii = jnp.broadcast_to(i, (LANES,)).astype(jnp.int32)
        plsc.store_scatter(ref, [ii], jnp.broadcast_to(val, (LANES,)),
                           mask=lane0)

    @pl.when(wid < B)
    def _work():
        b = wid
        with jax.named_scope("sc_stage_in"):
            stage = pltpu.async_copy(cost_hbm.at[b], cost_sh.at[b], dsem)
            hs = [pltpu.async_copy(colmin_hbm.at[b, 0], colmin_v, dsem2),
                  pltpu.async_copy(colargq_hbm.at[b, 0], colargq_v, dsem2),
                  pltpu.async_copy(pp_hbm.at[b], pp_v, dsem2),
                  pltpu.async_copy(tg_hbm.at[b], tg_v, dsem2),
                  pltpu.async_copy(size_hbm.at[b], size_v, dsem2)]
            for h in hs:
                h.wait()

        zeros16 = jnp.zeros((LANES,), jnp.float32)
        for k in range(QP // LANES):
            qmask_v[pl.ds(k * LANES, LANES)] = zeros16
        for k in range(MP // LANES):
            _put(summary_v, k, jnp.min(colmin_v[pl.ds(k * LANES, LANES)]))
        mqm_v[pl.ds(M - 8, LANES)] = jnp.zeros((LANES,), jnp.int32)

        sizes = size_v[pl.ds(0, LANES)]
        rcp = 1.0 / sizes.astype(jnp.float32)
        rw = rcp[1]
        rh = rcp[0]
        rs = (rw, rh, rw, rh)
        inf = jnp.float32(jnp.inf)

        def upd_summary(m):
            k = lax.shift_right_logical(m, 4)
            _put(summary_v, k, jnp.min(colmin_v[pl.ds(k * LANES, LANES)]))

        def recompute_col(m2):
            pltpu.sync_copy(cost_sh.at[b, m2], rowbuf_v)
            bv = rowbuf_v[pl.ds(0, LANES)] + qmask_v[pl.ds(0, LANES)]
            bq = iota16
            for k in range(1, QP // LANES):
                v = rowbuf_v[pl.ds(k * LANES, LANES)] + qmask_v[pl.ds(k * LANES, LANES)]
                qv = iota16 + (k * LANES)
                lt2 = (v < bv) | ((v == bv) & (qv < bq))
                bv = jnp.where(lt2, v, bv)
                bq = jnp.where(lt2, qv, bq)
            mv = jnp.min(bv)
            _put(colmin_v, m2, mv)
            _put(colargq_v, m2, jnp.min(jnp.where(bv == mv, bq, BIGI)))
            upd_summary(m2)

        def full_scan():
            bv = colmin_v[pl.ds(0, LANES)]
            bq = colargq_v[pl.ds(0, LANES)]
            bm = iota16
            for k in range(1, MP // LANES):
                v = colmin_v[pl.ds(k * LANES, LANES)]
                qv = colargq_v[pl.ds(k * LANES, LANES)]
                mv_ = iota16 + (k * LANES)
                lt2 = (v < bv) | ((v == bv) & ((qv < bq) | ((qv == bq) & (mv_ < bm))))
                bv = jnp.where(lt2, v, bv)
                bq = jnp.where(lt2, qv, bq)
                bm = jnp.where(lt2, mv_, bm)
            gv = jnp.min(bv)
            c1 = bv == gv
            gq = jnp.min(jnp.where(c1, bq, BIGI))
            gm = jnp.min(jnp.where(c1 & (bq == gq), bm, BIGI))
            return gv, gq, gm

        def scan_min():
            sv = summary_v[pl.ds(0, LANES)]
            sk, skidx = plsc.sort_key_val(sv, iota16)
            k0 = skidx[0]
            off = k0 * LANES
            v = colmin_v[pl.ds(off, LANES)]
            qv = colargq_v[pl.ds(off, LANES)]
            ck, cp = plsc.sort_key_val(v, qv * LANES + iota16)
            p0 = cp[0]
            tie = (sk[1] == sk[0]) | (ck[1] == ck[0])

            def slow():
                gv, gq, gm = full_scan()
                ks = lax.shift_right_logical(gm, 4)
                cv = colmin_v[pl.ds(ks * LANES, LANES)]
                cv = jnp.where(iota16 == (gm & (LANES - 1)), jnp.inf, cv)
                return gv, gq, gm, jnp.min(cv)

            return lax.cond(
                tie, slow,
                lambda: (ck[0], lax.shift_right_logical(p0, 4),
                         off + (p0 & (LANES - 1)), ck[1]))

        def body(it, carry):
            def wcond(st):
                return _gat(qmask_v, st[1]) != 0.0

            def wbody(st):
                recompute_col(st[2])
                return scan_min()

            gv, gq, gm, alt = lax.while_loop(wcond, wbody, scan_min())

            _put(mqm_v, it, gq * MP + gm)

            _put(colmin_v, gm, inf)
            _put(qmask_v, gq, inf)
            _put(summary_v, lax.shift_right_logical(gm, 4), alt)
            return carry

        with jax.named_scope("sc_match"):
            stage.wait()
            lax.fori_loop(0, M, body, 0, unroll=2)

        z16 = jnp.zeros((LANES,), jnp.float32)
        ccorr_v = z16
        bsum_v = z16
        asum_v = z16
        for g in range(MP // LANES):
            base_i = g * LANES
            if base_i >= M:
                break
            pk = mqm_v[pl.ds(base_i, LANES)]
            q16 = lax.shift_right_logical(pk, 8)
            m16 = pk & (MP - 1)
            c5 = jnp.broadcast_to(jnp.int32(5), (LANES,))
            labm16 = plsc.load_gather(tg_v, [m16, c5]).astype(jnp.int32) - 1
            lp = plsc.load_gather(pp_v, [labm16, q16])
            lp15 = plsc.load_gather(pp_v, [c5 + (NCLS - 5), q16])
            t = [plsc.load_gather(tg_v, [m16, jnp.broadcast_to(jnp.int32(d), (LANES,))])
                 for d in range(5)]
            p = [plsc.load_gather(pp_v, [jnp.broadcast_to(jnp.int32(NCLS + 1 + d), (LANES,)), q16])
                 for d in range(5)]
            cc = -lp + NOOBJ_W * lp15
            l1 = jnp.abs(p[0] - t[0] * rs[0])
            for d in range(1, 4):
                l1 = l1 + jnp.abs(p[d] - t[d] * rs[d])
            dth = p[4] - t[4]
            av = 1.0 - _cos_scalar(dth)
            if base_i + LANES > M:
                valid = iota16 < (M - base_i)
                cc = jnp.where(valid, cc, 0.0)
                l1 = jnp.where(valid, l1, 0.0)
                av = jnp.where(valid, av, 0.0)
            ccorr_v = ccorr_v + cc
            bsum_v = bsum_v + l1
            asum_v = asum_v + av
        ccorr = jnp.sum(ccorr_v)
        bsum = jnp.sum(bsum_v)
        asum = jnp.sum(asum_v)
        out16 = jnp.where(iota16 == 0, ccorr,
                          jnp.where(iota16 == 1, bsum,
                                    jnp.where(iota16 == 2, asum, 0.0)))
        outbuf_v[pl.ds(0, LANES)] = out16
        pltpu.sync_copy(outbuf_v, out_hbm.at[b])


def _sc_stage(cost, colmin, colargq, pp, tg, size):
    mesh = plsc.VectorSubcoreMesh(core_axis_name="c", subcore_axis_name="s")
    f32 = jnp.float32
    fn = functools.partial(
        pl.kernel,
        mesh=mesh,
        compiler_params=pltpu.CompilerParams(needs_layout_passes=False),
        out_type=jax.ShapeDtypeStruct((B, 16), f32),
        scratch_types=[
            pltpu.VMEM((MP,), f32),
            pltpu.VMEM((MP,), jnp.int32),
            pltpu.VMEM((NPP, QP), f32),
            pltpu.VMEM((M, 8), f32),
            pltpu.VMEM((LANES,), jnp.int32),
            pltpu.VMEM((QP,), f32),
            pltpu.VMEM((QP,), f32),
            pltpu.VMEM((LANES,), f32),
            pltpu.VMEM((LANES,), f32),
            pltpu.VMEM((MP,), jnp.int32),
            pltpu.VMEM_SHARED((B, M, QP), f32),
            pltpu.SemaphoreType.DMA,
            pltpu.SemaphoreType.DMA,
        ],
    )(_sc_greedy)
    return fn(cost, colmin, colargq, pp, tg, size)


@jax.jit
def kernel(pred_logits, pred_boxes, tgt_boxes, tgt_labels, tgt_size):
    f32 = jnp.float32
    size32 = tgt_size.astype(jnp.int32)
    sizep = jnp.pad(size32, ((0, 0), (0, 14)))

    cost, colmin, colargq, pp, tg, base = _tc_stage(
        size32[:, None, :],
        jnp.swapaxes(pred_logits.astype(f32), 1, 2),
        jnp.swapaxes(pred_boxes.astype(f32), 1, 2),
        tgt_boxes.astype(f32), tgt_labels.astype(jnp.int32)[..., None])

    res = _sc_stage(cost, colmin, colargq, pp, tg, sizep)

    denom = NOOBJ_W * (Q - M) + 1.0 * M
    loss_cls = jnp.mean((base[:, 0, 0] + res[:, 0]) / denom)
    loss_bbox = jnp.mean(res[:, 1] / (M * 4)) * BBOX_W
    loss_ang = jnp.mean(res[:, 2] / M) * ANG_W
    return (loss_cls + loss_bbox + loss_ang, loss_cls, loss_bbox, loss_ang)


# speedup vs baseline: 77.8359x; 1.0671x over previous
import functools

import jax
import jax.numpy as jnp
from jax import lax
from jax.experimental import pallas as pl
from jax.experimental.pallas import tpu as pltpu
from jax.experimental.pallas import tpu_sc as plsc

NCLS = 15
CLS_W = 2.0
BBOX_W = 5.0
ANG_W = 2.0
NOOBJ_W = 0.1
B, Q, M = 4, 1000, 200
QP, MP = 1024, 256
NPP = NCLS + 1 + 5
LANES = 16
BIGI = 2 ** 30

_COS_C = [
    1.0, -0.5, 1.0 / 24, -1.0 / 720, 1.0 / 40320, -1.0 / 3628800,
    1.0 / 479001600, -1.0 / 87178291200,
]


def _cos_scalar(x):
    t = x * x
    r = jnp.float32(_COS_C[7])
    for k in range(6, -1, -1):
        r = r * t + jnp.float32(_COS_C[k])
    return r


def _tc_body(size_ref, lg_ref, pb_ref, tb_ref, lab_ref,
             cost_ref, colmin_ref, colargq_ref, pp_ref, tg_ref, base_ref):
    zq = jnp.zeros((16, QP - Q), jnp.float32)
    lt = jnp.concatenate([lg_ref[0], zq], axis=1)
    pbt = jnp.concatenate([pb_ref[0], zq[:5]], axis=1)
    mx = jnp.max(lt, axis=0, keepdims=True)
    ex = jnp.exp(lt - mx)
    s = jnp.sum(ex, axis=0, keepdims=True)
    logp = lt - mx - jnp.log(s)
    pp_ref[0, :NCLS + 1, :] = logp
    pp_ref[0, NCLS + 1:, :] = pbt
    prob = ex / s

    hh = size_ref[0, 0, 0].astype(jnp.float32)
    ww = size_ref[0, 0, 1].astype(jnp.float32)
    bidx = pl.program_id(0)
    tbr = tb_ref[bidx]
    labr = jnp.swapaxes(lab_ref[pl.ds(bidx, 1), :], 0, 1)
    tg_ref[0, :, :5] = tbr
    tg_ref[0, :, 5:6] = labr.astype(jnp.float32)
    tg_ref[0, :, 6:] = jnp.zeros((M, 2), jnp.float32)
    tb = jnp.concatenate([tbr, jnp.zeros((MP - M, 5), jnp.float32)], axis=0)
    lab0 = jnp.concatenate(
        [labr - 1, jnp.full((MP - M, 1), NCLS, jnp.int32)], axis=0)

    sel = [prob[c:c + 1, :] for c in range(16)]
    for bit in (1, 2, 4, 8):
        cond = (lab0 & bit) != 0
        sel = [jnp.where(cond, sel[i + 1], sel[i])
               for i in range(0, len(sel), 2)]
    cls_cost = sel[0] * (-CLS_W)

    di = lax.broadcasted_iota(jnp.int32, (1, 5), 1)
    scale5 = jnp.where((di == 0) | (di == 2), ww,
                       jnp.where(di == 4, 1.0, hh))
    tbn = tb / scale5
    l1 = jnp.abs(pbt[0:1, :] - tbn[:, 0:1])
    for d in range(1, 4):
        l1 = l1 + jnp.abs(pbt[d:d + 1, :] - tbn[:, d:d + 1])
    pth = pbt[4:5, :]
    tth = tb[:, 4:5]
    ang = 1.0 - (jnp.cos(pth) * jnp.cos(tth) + jnp.sin(pth) * jnp.sin(tth))
    cost = cls_cost + l1 * BBOX_W + ang * ANG_W

    qi = lax.broadcasted_iota(jnp.int32, (MP, QP), 1)
    mi = lax.broadcasted_iota(jnp.int32, (MP, QP), 0)
    cost = jnp.where((qi >= Q) | (mi >= M), jnp.inf, cost)
    cost_ref[0] = cost[:M]

    cmin = jnp.min(cost, axis=1, keepdims=True)
    colmin_ref[0] = jnp.swapaxes(cmin, 0, 1)
    ismin = cost == cmin
    argq = jnp.min(jnp.where(ismin, qi, QP), axis=1, keepdims=True)
    colargq_ref[0] = jnp.swapaxes(argq, 0, 1)

    row15 = logp[NCLS:NCLS + 1, :]
    qrow = lax.broadcasted_iota(jnp.int32, (1, QP), 1)
    base_ref[0, 0, 0] = NOOBJ_W * jnp.sum(jnp.where(qrow < Q, -row15, 0.0))


def _tc_stage(size, lg, pb, tbr, labr):
    f32 = jnp.float32
    out_shapes = (
        jax.ShapeDtypeStruct((B, M, QP), f32),
        jax.ShapeDtypeStruct((B, 1, MP), f32),
        jax.ShapeDtypeStruct((B, 1, MP), jnp.int32),
        jax.ShapeDtypeStruct((B, NPP, QP), f32),
        jax.ShapeDtypeStruct((B, M, 8), f32),
        jax.ShapeDtypeStruct((B, 1, 1), f32),
    )
    grid = (B,)
    return pl.pallas_call(
        _tc_body,
        grid=grid,
        in_specs=[
            pl.BlockSpec((1, 1, 2), lambda b: (b, 0, 0), memory_space=pltpu.SMEM),
            pl.BlockSpec((1, 16, Q), lambda b: (b, 0, 0)),
            pl.BlockSpec((1, 5, Q), lambda b: (b, 0, 0)),
            pl.BlockSpec((B, M, 5), lambda b: (0, 0, 0)),
            pl.BlockSpec((B, M), lambda b: (0, 0)),
        ],
        out_specs=[
            pl.BlockSpec((1, M, QP), lambda b: (b, 0, 0)),
            pl.BlockSpec((1, 1, MP), lambda b: (b, 0, 0)),
            pl.BlockSpec((1, 1, MP), lambda b: (b, 0, 0)),
            pl.BlockSpec((1, NPP, QP), lambda b: (b, 0, 0)),
            pl.BlockSpec((1, M, 8), lambda b: (b, 0, 0)),
            pl.BlockSpec((1, 1, 1), lambda b: (b, 0, 0), memory_space=pltpu.SMEM),
        ],
        out_shape=out_shapes,
    )(size, lg, pb, tbr, labr)


def _sc_greedy(cost_hbm, colmin_hbm, colargq_hbm, pp_hbm, tg_hbm, size_hbm,
               out_hbm,
               colmin_v, colargq_v, pp_v, tg_v, size_v,
               qmask_v, rowbuf_v, outbuf_v, summary_v, mqm_v, cost_sh, dsem,
               dsem2):
    info = plsc.get_sparse_core_info()
    ns = info.num_subcores
    wid = lax.axis_index("c") * ns + lax.axis_index("s")

    iota16 = lax.broadcasted_iota(jnp.int32, (LANES,), 0)
    lane0 = iota16 == 0

    def _gat(ref, *idx):
        idxs = [jnp.broadcast_to(i, (LANES,)).astype(jnp.int32) for i in idx]
        return plsc.load_gather(ref, idxs)[0]

    def _gatv(ref, idx16):
        return plsc.load_gather(ref, [idx16])

    def _put(ref, i, val):
        ii = jnp.broadcast_to(i, (LANES,)).astype(jnp.int32)
        plsc.store_scatter(ref, [ii], jnp.broadcast_to(val, (LANES,)),
                           mask=lane0)

    @pl.when(wid < B)
    def _work():
        b = wid
        with jax.named_scope("sc_stage_in"):
            stage = pltpu.async_copy(cost_hbm.at[b], cost_sh.at[b], dsem)
            hs = [pltpu.async_copy(colmin_hbm.at[b, 0], colmin_v, dsem2),
                  pltpu.async_copy(colargq_hbm.at[b, 0], colargq_v, dsem2),
                  pltpu.async_copy(pp_hbm.at[b], pp_v, dsem2),
                  pltpu.async_copy(tg_hbm.at[b], tg_v, dsem2),
                  pltpu.async_copy(size_hbm.at[b], size_v, dsem2)]
            for h in hs:
                h.wait()

        zeros16 = jnp.zeros((LANES,), jnp.float32)
        for k in range(QP // LANES):
            qmask_v[pl.ds(k * LANES, LANES)] = zeros16
        for k in range(MP // LANES):
            _put(summary_v, k, jnp.min(colmin_v[pl.ds(k * LANES, LANES)]))
        mqm_v[pl.ds(M - 8, LANES)] = jnp.zeros((LANES,), jnp.int32)

        sizes = size_v[pl.ds(0, LANES)]
        rcp = 1.0 / sizes.astype(jnp.float32)
        rw = rcp[1]
        rh = rcp[0]
        rs = (rw, rh, rw, rh)
        inf = jnp.float32(jnp.inf)

        def upd_summary(m):
            k = lax.shift_right_logical(m, 4)
            _put(summary_v, k, jnp.min(colmin_v[pl.ds(k * LANES, LANES)]))

        def recompute_col(m2):
            pltpu.sync_copy(cost_sh.at[b, m2], rowbuf_v)
            bv = rowbuf_v[pl.ds(0, LANES)] + qmask_v[pl.ds(0, LANES)]
            bq = iota16
            for k in range(1, QP // LANES):
                v = rowbuf_v[pl.ds(k * LANES, LANES)] + qmask_v[pl.ds(k * LANES, LANES)]
                qv = iota16 + (k * LANES)
                lt2 = (v < bv) | ((v == bv) & (qv < bq))
                bv = jnp.where(lt2, v, bv)
                bq = jnp.where(lt2, qv, bq)
            mv = jnp.min(bv)
            _put(colmin_v, m2, mv)
            _put(colargq_v, m2, jnp.min(jnp.where(bv == mv, bq, BIGI)))
            upd_summary(m2)

        def full_scan():
            bv = colmin_v[pl.ds(0, LANES)]
            bq = colargq_v[pl.ds(0, LANES)]
            bm = iota16
            for k in range(1, MP // LANES):
                v = colmin_v[pl.ds(k * LANES, LANES)]
                qv = colargq_v[pl.ds(k * LANES, LANES)]
                mv_ = iota16 + (k * LANES)
                lt2 = (v < bv) | ((v == bv) & ((qv < bq) | ((qv == bq) & (mv_ < bm))))
                bv = jnp.where(lt2, v, bv)
                bq = jnp.where(lt2, qv, bq)
                bm = jnp.where(lt2, mv_, bm)
            gv = jnp.min(bv)
            c1 = bv == gv
            gq = jnp.min(jnp.where(c1, bq, BIGI))
            gm = jnp.min(jnp.where(c1 & (bq == gq), bm, BIGI))
            return gv, gq, gm

        def full_scan_alt():
            gv, gq, gm = full_scan()
            ks = lax.shift_right_logical(gm, 4)
            cv = colmin_v[pl.ds(ks * LANES, LANES)]
            cv = jnp.where(iota16 == (gm & (LANES - 1)), jnp.inf, cv)
            return gv, gq, gm, jnp.min(cv)

        def scan_raw():
            sv = summary_v[pl.ds(0, LANES)]
            sk, skidx = plsc.sort_key_val(sv, iota16)
            k0 = skidx[0]
            off = k0 * LANES
            v = colmin_v[pl.ds(off, LANES)]
            qv = colargq_v[pl.ds(off, LANES)]
            ck, cp = plsc.sort_key_val(v, qv * LANES + iota16)
            p0 = cp[0]
            tie = (sk[1] == sk[0]) | (ck[1] == ck[0])
            return (tie, ck[0], lax.shift_right_logical(p0, 4),
                    off + (p0 & (LANES - 1)), ck[1])

        def body(it, carry):
            tie, v0, q0, m0, a0 = scan_raw()
            stale = _gat(qmask_v, q0) != 0.0

            def slowpath():
                st = lax.cond(tie, full_scan_alt,
                              lambda: (v0, q0, m0, a0))

                def wcond(s):
                    return _gat(qmask_v, s[1]) != 0.0

                def wbody(s):
                    recompute_col(s[2])
                    t2, v2, q2, m2, a2 = scan_raw()
                    return lax.cond(t2, full_scan_alt,
                                    lambda: (v2, q2, m2, a2))

                return lax.while_loop(wcond, wbody, st)

            gv, gq, gm, alt = lax.cond(tie | stale, slowpath,
                                       lambda: (v0, q0, m0, a0))

            _put(mqm_v, it, gq * MP + gm)

            _put(colmin_v, gm, inf)
            _put(qmask_v, gq, inf)
            _put(summary_v, lax.shift_right_logical(gm, 4), alt)
            return carry

        with jax.named_scope("sc_match"):
            stage.wait()
            lax.fori_loop(0, M, body, 0, unroll=2)

        z16 = jnp.zeros((LANES,), jnp.float32)
        ccorr_v = z16
        bsum_v = z16
        asum_v = z16
        for g in range(MP // LANES):
            base_i = g * LANES
            if base_i >= M:
                break
            pk = mqm_v[pl.ds(base_i, LANES)]
            q16 = lax.shift_right_logical(pk, 8)
            m16 = pk & (MP - 1)
            c5 = jnp.broadcast_to(jnp.int32(5), (LANES,))
            labm16 = plsc.load_gather(tg_v, [m16, c5]).astype(jnp.int32) - 1
            lp = plsc.load_gather(pp_v, [labm16, q16])
            lp15 = plsc.load_gather(pp_v, [c5 + (NCLS - 5), q16])
            t = [plsc.load_gather(tg_v, [m16, jnp.broadcast_to(jnp.int32(d), (LANES,))])
                 for d in range(5)]
            p = [plsc.load_gather(pp_v, [jnp.broadcast_to(jnp.int32(NCLS + 1 + d), (LANES,)), q16])
                 for d in range(5)]
            cc = -lp + NOOBJ_W * lp15
            l1 = jnp.abs(p[0] - t[0] * rs[0])
            for d in range(1, 4):
                l1 = l1 + jnp.abs(p[d] - t[d] * rs[d])
            dth = p[4] - t[4]
            av = 1.0 - _cos_scalar(dth)
            if base_i + LANES > M:
                valid = iota16 < (M - base_i)
                cc = jnp.where(valid, cc, 0.0)
                l1 = jnp.where(valid, l1, 0.0)
                av = jnp.where(valid, av, 0.0)
            ccorr_v = ccorr_v + cc
            bsum_v = bsum_v + l1
            asum_v = asum_v + av
        ccorr = jnp.sum(ccorr_v)
        bsum = jnp.sum(bsum_v)
        asum = jnp.sum(asum_v)
        out16 = jnp.where(iota16 == 0, ccorr,
                          jnp.where(iota16 == 1, bsum,
                                    jnp.where(iota16 == 2, asum, 0.0)))
        outbuf_v[pl.ds(0, LANES)] = out16
        pltpu.sync_copy(outbuf_v, out_hbm.at[b])


def _sc_stage(cost, colmin, colargq, pp, tg, size):
    mesh = plsc.VectorSubcoreMesh(core_axis_name="c", subcore_axis_name="s")
    f32 = jnp.float32
    fn = functools.partial(
        pl.kernel,
        mesh=mesh,
        compiler_params=pltpu.CompilerParams(needs_layout_passes=False),
        out_type=jax.ShapeDtypeStruct((B, 16), f32),
        scratch_types=[
            pltpu.VMEM((MP,), f32),
            pltpu.VMEM((MP,), jnp.int32),
            pltpu.VMEM((NPP, QP), f32),
            pltpu.VMEM((M, 8), f32),
            pltpu.VMEM((LANES,), jnp.int32),
            pltpu.VMEM((QP,), f32),
            pltpu.VMEM((QP,), f32),
            pltpu.VMEM((LANES,), f32),
            pltpu.VMEM((LANES,), f32),
            pltpu.VMEM((MP,), jnp.int32),
            pltpu.VMEM_SHARED((B, M, QP), f32),
            pltpu.SemaphoreType.DMA,
            pltpu.SemaphoreType.DMA,
        ],
    )(_sc_greedy)
    return fn(cost, colmin, colargq, pp, tg, size)


@jax.jit
def kernel(pred_logits, pred_boxes, tgt_boxes, tgt_labels, tgt_size):
    f32 = jnp.float32
    size32 = tgt_size.astype(jnp.int32)
    sizep = jnp.pad(size32, ((0, 0), (0, 14)))

    cost, colmin, colargq, pp, tg, base = _tc_stage(
        size32[:, None, :],
        jnp.swapaxes(pred_logits.astype(f32), 1, 2),
        jnp.swapaxes(pred_boxes.astype(f32), 1, 2),
        tgt_boxes.astype(f32), tgt_labels.astype(jnp.int32))

    res = _sc_stage(cost, colmin, colargq, pp, tg, sizep)

    denom = NOOBJ_W * (Q - M) + 1.0 * M
    loss_cls = jnp.mean((base[:, 0, 0] + res[:, 0]) / denom)
    loss_bbox = jnp.mean(res[:, 1] / (M * 4)) * BBOX_W
    loss_ang = jnp.mean(res[:, 2] / M) * ANG_W
    return (loss_cls + loss_bbox + loss_ang, loss_cls, loss_bbox, loss_ang)
